# Initial kernel scaffold; baseline (speedup 1.0000x reference)
import jax
import jax.numpy as jnp
from jax.experimental import pallas as pl


def kernel(q, mu, r_ij, d_ij, mlp_in_w, mlp_in_b, mlp_out_w1, mlp_out_b1, mlp_out_w2, mlp_out_b2, sem_w, sem_b, xmix_w, post_w1, post_b1, post_w2, post_b2, node_w1, node_b1, node_w2, node_b2, idx_i, idx_j):
    raise NotImplementedError("write your pallas kernel here")


# speedup vs baseline: 5.1293x; 5.1293x over previous
import functools

import jax
import jax.numpy as jnp
from jax.experimental import pallas as pl

N_ATOMS = 10000
N_PAIRS = 160000
F = 128
H = 2
C = H * F
N_RBF = 50
RBF_PAD = 64
CUTOFF = 5.0

E_BLK = 2000
N_BLK = 2000

_INTERPRET = False


def _silu(x):
    return x * jax.nn.sigmoid(x)


def _celu2(x):
    return jnp.where(x > 0, x, 2.0 * (jnp.exp(x * 0.5) - 1.0))


def _edge1_body(qi, qj, d, wa_t, wb_t, b_in, offs, coeff_s, w1i_t, w1j_t,
                w1f_t, w1d, b1, w2_t, b2, sem_t, semb, m_out, e_out):
    qi_v = qi[...]
    qj_v = qj[...]
    d_v = d[...]
    qe = qi_v @ wa_t[...] + qj_v @ wb_t[...] + b_in[...]
    rbf = jnp.exp(coeff_s[0, 0] * (d_v - offs[...]) ** 2)
    filt = rbf * qe
    pre = (qi_v @ w1i_t[...] + qj_v @ w1j_t[...] + filt @ w1f_t[...]
           + d_v * w1d[...] + b1[...])
    h = _silu(pre)
    m = h @ w2_t[...] + b2[...]
    att = _celu2(m @ sem_t[...] + semb[...])
    m_out[...] = m
    e_out[...] = jnp.exp(jnp.minimum(att, 60.0))


def _edge2_body(m, e, ssg, r, d, x0_t, x1_t, acat_out, coeff_out, combos_out):
    m_v = m[...]
    comb = e[...] / ssg[...]
    a0 = m_v * comb[:, 0:1]
    a1 = m_v * comb[:, 1:2]
    coeff = jnp.tanh(a0 @ x0_t[...] + a1 @ x1_t[...])
    rhat = r[...] / (d[...] + 1e-05)
    acat_out[...] = jnp.concatenate([a0, a1], axis=1)
    coeff_out[...] = coeff
    combos_out[...] = jnp.concatenate(
        [coeff * rhat[:, 0:1], coeff * rhat[:, 1:2], coeff * rhat[:, 2:3]],
        axis=1)


def _node_body(q, agg, cs, cnt, pw1_t, pb1, pw2_t, pb2, wq_t, wa_t, wc_t,
               nb1, nw2_t, nb2, out):
    q_v = q[...]
    cnt_v = jnp.maximum(cnt[...], 1.0)
    mean = cs[...] / cnt_v
    n0 = mean[:, 0:C]
    n1 = mean[:, C:2 * C]
    n2 = mean[:, 2 * C:3 * C]
    norm = n0 * n0 + n1 * n1 + n2 * n2
    qc = _silu(norm @ pw1_t[...] + pb1[...])
    qcomb = _silu(qc @ pw2_t[...] + pb2[...])
    h = _silu(q_v @ wq_t[...] + agg[...] @ wa_t[...] + qcomb @ wc_t[...]
              + nb1[...])
    out[...] = q_v + _silu(h @ nw2_t[...] + nb2[...])


def _full_spec(shape):
    return pl.BlockSpec(shape, lambda i: (0,) * len(shape))


def _row_spec(blk, width):
    return pl.BlockSpec((blk, width), lambda i: (i, 0))


def kernel(q, mu, r_ij, d_ij, mlp_in_w, mlp_in_b, mlp_out_w1, mlp_out_b1,
           mlp_out_w2, mlp_out_b2, sem_w, sem_b, xmix_w, post_w1, post_b1,
           post_w2, post_b2, node_w1, node_b1, node_w2, node_b2, idx_i, idx_j):
    n_atoms = q.shape[0]
    n_pairs = idx_i.shape[0]

    f32 = jnp.float32
    pad_rbf = lambda a, axis: jnp.concatenate(
        [a, jnp.zeros(a.shape[:axis] + (RBF_PAD - N_RBF,) + a.shape[axis + 1:],
                      f32)], axis=axis)
    wa_t = pad_rbf(mlp_in_w[:, :F].T, 1)
    wb_t = pad_rbf(mlp_in_w[:, F:].T, 1)
    b_in = pad_rbf(mlp_in_b[None, :], 1)
    offsets = jnp.linspace(0.0, CUTOFF, N_RBF)
    offs = pad_rbf(offsets[None, :], 1)
    width = offsets[1] - offsets[0]
    coeff_s = jnp.full((1, 1), -0.5 / (width ** 2), f32)
    w1i_t = mlp_out_w1[:, :F].T
    w1j_t = mlp_out_w1[:, F:2 * F].T
    w1f_t = pad_rbf(mlp_out_w1[:, 2 * F:2 * F + N_RBF], 1).T
    w1d = mlp_out_w1[:, 2 * F + N_RBF][None, :]
    b1 = mlp_out_b1[None, :]
    w2_t = mlp_out_w2.T
    b2 = mlp_out_b2[None, :]
    sem_t = sem_w.T
    semb = sem_b[None, :]
    xm = xmix_w.reshape(C, F, H)
    x0_t = xm[:, :, 0].T
    x1_t = xm[:, :, 1].T
    pw1_t = post_w1.T
    pb1 = post_b1[None, :]
    pw2_t = post_w2.T
    pb2 = post_b2[None, :]
    nq_t = node_w1[:, :F].T
    na_perm = node_w1[:, F:F + C].reshape(F, F, H).transpose(2, 1, 0)
    na_t = na_perm.reshape(C, F)
    nc_t = node_w1[:, F + C:].T
    nb1 = node_b1[None, :]
    nw2_t = node_w2.T
    nb2 = node_b2[None, :]

    d2 = d_ij.astype(f32)

    qi = q[idx_i]
    qj = q[idx_j]

    grid_e = n_pairs // E_BLK
    m, e = pl.pallas_call(
        _edge1_body,
        grid=(grid_e,),
        in_specs=[
            _row_spec(E_BLK, F), _row_spec(E_BLK, F), _row_spec(E_BLK, 1),
            _full_spec((F, RBF_PAD)), _full_spec((F, RBF_PAD)),
            _full_spec((1, RBF_PAD)), _full_spec((1, RBF_PAD)),
            _full_spec((1, 1)),
            _full_spec((F, F)), _full_spec((F, F)), _full_spec((RBF_PAD, F)),
            _full_spec((1, F)), _full_spec((1, F)), _full_spec((F, F)),
            _full_spec((1, F)), _full_spec((F, H)), _full_spec((1, H)),
        ],
        out_specs=[_row_spec(E_BLK, F), _row_spec(E_BLK, H)],
        out_shape=[
            jax.ShapeDtypeStruct((n_pairs, F), f32),
            jax.ShapeDtypeStruct((n_pairs, H), f32),
        ],
        interpret=_INTERPRET,
    )(qi, qj, d2, wa_t, wb_t, b_in, offs, coeff_s, w1i_t, w1j_t, w1f_t, w1d,
      b1, w2_t, b2, sem_t, semb)

    seg_sum = jax.ops.segment_sum(e, idx_j, num_segments=n_atoms)
    ssg = seg_sum[idx_j]

    acat, coeff, combos = pl.pallas_call(
        _edge2_body,
        grid=(grid_e,),
        in_specs=[
            _row_spec(E_BLK, F), _row_spec(E_BLK, H), _row_spec(E_BLK, H),
            _row_spec(E_BLK, 3), _row_spec(E_BLK, 1),
            _full_spec((F, C)), _full_spec((F, C)),
        ],
        out_specs=[_row_spec(E_BLK, C), _row_spec(E_BLK, C),
                   _row_spec(E_BLK, 3 * C)],
        out_shape=[
            jax.ShapeDtypeStruct((n_pairs, C), f32),
            jax.ShapeDtypeStruct((n_pairs, C), f32),
            jax.ShapeDtypeStruct((n_pairs, 3 * C), f32),
        ],
        interpret=_INTERPRET,
    )(m, e, ssg, r_ij, d2, x0_t, x1_t)

    agg = jax.ops.segment_sum(acat, idx_j, num_segments=n_atoms)
    comb_sum = jax.ops.segment_sum(combos, idx_j, num_segments=n_atoms)
    counts = jax.ops.segment_sum(jnp.ones((n_pairs, 1), f32), idx_j,
                                 num_segments=n_atoms)

    grid_n = n_atoms // N_BLK
    out = pl.pallas_call(
        _node_body,
        grid=(grid_n,),
        in_specs=[
            _row_spec(N_BLK, F), _row_spec(N_BLK, C), _row_spec(N_BLK, 3 * C),
            _row_spec(N_BLK, 1),
            _full_spec((C, F)), _full_spec((1, F)), _full_spec((F, F)),
            _full_spec((1, F)), _full_spec((F, F)), _full_spec((C, F)),
            _full_spec((F, F)), _full_spec((1, F)), _full_spec((F, F)),
            _full_spec((1, F)),
        ],
        out_specs=[_row_spec(N_BLK, F)],
        out_shape=[jax.ShapeDtypeStruct((n_atoms, F), f32)],
        interpret=_INTERPRET,
    )(q, agg, comb_sum, counts, pw1_t, pb1, pw2_t, pb2, nq_t, na_t, nc_t,
      nb1, nw2_t, nb2)[0]
    return out


# speedup vs baseline: 6.0110x; 1.1719x over previous
import functools

import jax
import jax.numpy as jnp
from jax import lax
from jax.experimental import pallas as pl
from jax.experimental.pallas import tpu as pltpu
from jax.experimental.pallas import tpu_sc as plsc

N_ATOMS = 10000
N_PAIRS = 160000
F = 128
H = 2
C = H * F
N_RBF = 50
RBF_PAD = 64
CUTOFF = 5.0

E_BLK = 2048
N_BLK = 1000
E_P = 163840
CHUNK = 1024
N_T = 10112
ROWS_T = N_T // 16

_INTERPRET = False


def _silu(x):
    return x * jax.nn.sigmoid(x)


def _celu2(x):
    return jnp.where(x > 0, x, 2.0 * (jnp.exp(x * 0.5) - 1.0))


def _mesh():
    return plsc.VectorSubcoreMesh(core_axis_name="c", subcore_axis_name="s")


_GDN = lax.GatherDimensionNumbers(
    offset_dims=(), collapsed_slice_dims=(0,), start_index_map=(0,))


def _sc_gather(table, idx2d, B, D):
    per_w = B // 32
    iters = per_w // CHUNK

    def body(table_h, idx_h, out_h, idx_v, rows_v, sem):
        cid = lax.axis_index("c")
        sid = lax.axis_index("s")
        wid = sid * 2 + cid

        def step(i, carry):
            off = wid * per_w + i * CHUNK
            r0 = wid * (per_w // 128) + i * 8
            pltpu.sync_copy(idx_h.at[pl.ds(r0, 8)], idx_v)
            for qq in range(4):
                descs = [
                    pltpu.async_copy(table_h.at[idx_v.at[qq * 2 + s]],
                                     rows_v.at[pl.ds(s * 128, 128)], sem)
                    for s in range(2)
                ]
                for dsc in descs:
                    dsc.wait()
                pltpu.sync_copy(rows_v, out_h.at[pl.ds(off + qq * 256, 256)])
            return carry

        lax.fori_loop(0, iters, step, 0)

    return pl.kernel(
        body,
        out_type=jax.ShapeDtypeStruct((B, D), jnp.float32),
        mesh=_mesh(),
        scratch_types=[
            pltpu.VMEM((8, 128), jnp.int32),
            pltpu.VMEM((256, D), jnp.float32),
            pltpu.SemaphoreType.DMA,
        ],
    )(table, idx2d)


def _sc_scatter_seg(vals, idx1d, zeros128):
    per_sc = E_P // 2
    per_t = per_sc // 16
    iters = per_t // 128

    def body(vals_h, idx_h, z_h, out_h, idx_v, rows_v, table):
        cid = lax.axis_index("c")
        sid = lax.axis_index("s")

        @pl.when(sid == 0)
        def _():
            pltpu.sync_copy(z_h, table)

        plsc.subcore_barrier()

        def step(i, carry):
            off = cid * per_sc + sid * per_t + i * 128
            pltpu.sync_copy(vals_h.at[pl.ds(off, 128)], rows_v)
            pltpu.sync_copy(idx_h.at[pl.ds(off, 128)], idx_v)
            pltpu.sync_copy(rows_v, table.at[idx_v], add=True)
            return carry

        lax.fori_loop(0, iters, step, 0)
        plsc.subcore_barrier()

        @pl.when(sid == 0)
        def _():
            pltpu.sync_copy(table, out_h.at[cid])

    return pl.kernel(
        body,
        out_type=jax.ShapeDtypeStruct((2, N_T, 128), jnp.float32),
        mesh=_mesh(),
        scratch_types=[
            pltpu.VMEM((128,), jnp.int32),
            pltpu.VMEM((128, 128), jnp.float32),
            pltpu.VMEM_SHARED((N_T, 128), jnp.float32),
        ],
    )(vals, idx1d, zeros128)


def _sc_scatter_acat(vals, idx1d, zeros128):
    per_sc = E_P // 2
    per_t = per_sc // 16
    iters = per_t // 128

    def body(vals_h, idx_h, z_h, out_h, idx_v, rows_v, table):
        cid = lax.axis_index("c")
        sid = lax.axis_index("s")
        for ch in range(2):
            @pl.when(sid == 0)
            def _():
                pltpu.sync_copy(z_h, table)

            plsc.subcore_barrier()

            def step(i, carry):
                off = cid * per_sc + sid * per_t + i * 128
                pltpu.sync_copy(vals_h.at[pl.ds(off, 128),
                                          pl.ds(ch * 128, 128)], rows_v)
                pltpu.sync_copy(idx_h.at[pl.ds(off, 128)], idx_v)
                pltpu.sync_copy(rows_v, table.at[idx_v], add=True)
                return carry

            lax.fori_loop(0, iters, step, 0)
            plsc.subcore_barrier()

            @pl.when(sid == 0)
            def _():
                pltpu.sync_copy(table, out_h.at[cid, ch])

            plsc.subcore_barrier()

    return pl.kernel(
        body,
        out_type=jax.ShapeDtypeStruct((2, 2, N_T, 128), jnp.float32),
        mesh=_mesh(),
        scratch_types=[
            pltpu.VMEM((128,), jnp.int32),
            pltpu.VMEM((128, 128), jnp.float32),
            pltpu.VMEM_SHARED((N_T, 128), jnp.float32),
        ],
    )(vals, idx1d, zeros128)


def _sc_scatter_combos(coeff2, rhat3, idx1d, zeros128):
    per_sc = E_P // 2
    per_t = per_sc // 16
    iters = per_t // 128

    def body(c0_h, c1_h, r0_h, r1_h, r2_h, idx_h, z_h, out_h, idx_v, rows_v,
             srows_v, rfac_v, table):
        cid = lax.axis_index("c")
        sid = lax.axis_index("s")
        coeff_hs = [c0_h, c1_h]
        rhat_hs = [r0_h, r1_h, r2_h]
        for x in range(3):
            for ch in range(2):
                @pl.when(sid == 0)
                def _():
                    pltpu.sync_copy(z_h, table)

                plsc.subcore_barrier()

                def step(i, carry):
                    o2 = cid * per_sc + sid * per_t + i * 128
                    pltpu.sync_copy(coeff_hs[ch].at[pl.ds(o2, 128)], rows_v)
                    pltpu.sync_copy(idx_h.at[pl.ds(o2, 128)], idx_v)
                    pltpu.sync_copy(rhat_hs[x].at[pl.ds(o2, 128)], rfac_v)

                    def gstep(g, cc):
                        base16 = (g // 2) * 16
                        half = (g % 2) * 8
                        rx = rfac_v[pl.ds(base16, 16)]
                        for j in range(8):
                            row = base16 + half + j
                            sidx = (jnp.zeros((16, 1), jnp.int32)
                                    + (half + j))
                            sc = lax.gather(
                                rx, sidx, _GDN, (1,),
                                mode=lax.GatherScatterMode
                                .PROMISE_IN_BOUNDS)
                            for v in range(8):
                                srows_v[row, pl.ds(v * 16, 16)] = (
                                    rows_v[row, pl.ds(v * 16, 16)] * sc)
                        return cc

                    lax.fori_loop(0, 16, gstep, 0)
                    pltpu.sync_copy(srows_v, table.at[idx_v], add=True)
                    return carry

                lax.fori_loop(0, iters, step, 0)
                plsc.subcore_barrier()

                @pl.when(sid == 0)
                def _():
                    pltpu.sync_copy(table, out_h.at[cid, x * 2 + ch])

                plsc.subcore_barrier()

    return pl.kernel(
        body,
        out_type=jax.ShapeDtypeStruct((2, 6, N_T, 128), jnp.float32),
        mesh=_mesh(),
        compiler_params=pltpu.CompilerParams(needs_layout_passes=False),
        scratch_types=[
            pltpu.VMEM((128,), jnp.int32),
            pltpu.VMEM((128, 128), jnp.float32),
            pltpu.VMEM((128, 128), jnp.float32),
            pltpu.VMEM((128,), jnp.float32),
            pltpu.VMEM_SHARED((N_T, 128), jnp.float32),
        ],
    )(*coeff2, *rhat3, idx1d, zeros128)


def _edge1_body(n_valid, qi, qj, d, wa_t, wb_t, b_in, offs, coeff_s, w1i_t,
                w1j_t, w1f_t, w1d, b1, w2_t, b2, sem_t, semb, m_out, e_out):
    qi_v = qi[...]
    qj_v = qj[...]
    d_v = d[...]
    qe = qi_v @ wa_t[...] + qj_v @ wb_t[...] + b_in[...]
    rbf = jnp.exp(coeff_s[0, 0] * (d_v - offs[...]) ** 2)
    filt = rbf * qe
    pre = (qi_v @ w1i_t[...] + qj_v @ w1j_t[...] + filt @ w1f_t[...]
           + d_v * w1d[...] + b1[...])
    h = _silu(pre)
    m = h @ w2_t[...] + b2[...]
    att = _celu2(m @ sem_t[...] + semb[...])
    e = jnp.exp(jnp.minimum(att, 60.0))
    rows = (pl.program_id(0) * E_BLK
            + lax.broadcasted_iota(jnp.int32, (E_BLK, 1), 0))
    valid = rows < n_valid
    e128 = jnp.concatenate(
        [e, jnp.ones((E_BLK, 1), jnp.float32),
         jnp.zeros((E_BLK, 125), jnp.float32)], axis=1)
    m_out[...] = m
    e_out[...] = jnp.where(valid, e128, 0.0)


def _edge2_body(m, e128, ssg128, r, d, x0_t, x1_t, acat_out, c0_out, c1_out,
                r0_out, r1_out, r2_out):
    m_v = m[...]
    e = e128[...][:, 0:H]
    ssg = jnp.maximum(ssg128[...][:, 0:H], 1e-30)
    comb = e / ssg
    a0 = m_v * comb[:, 0:1]
    a1 = m_v * comb[:, 1:2]
    coeff = jnp.tanh(a0 @ x0_t[...] + a1 @ x1_t[...])
    acat_out[...] = jnp.concatenate([a0, a1], axis=1)
    c0_out[...] = coeff[:, 0:F]
    c1_out[...] = coeff[:, F:C]
    rhat = r[...] / (d[...] + 1e-05)
    r0_out[...] = rhat[:, 0:1]
    r1_out[...] = rhat[:, 1:2]
    r2_out[...] = rhat[:, 2:3]


def _node_body(*refs):
    (q, agg0a, agg0b, agg1a, agg1b), cs_refs = refs[0:5], refs[5:17]
    (cnt128, pw1_t, pb1, pw2_t, pb2, wq_t, wa_t, wc_t, nb1, nw2_t, nb2,
     out) = refs[17:]
    q_v = q[...]
    agg = jnp.concatenate(
        [agg0a[...][0, 0] + agg1a[...][0, 0],
         agg0b[...][0, 0] + agg1b[...][0, 0]], axis=1)
    cnt_v = jnp.maximum(cnt128[...][:, 2:3], 1.0)
    norm_parts = []
    for ch in range(2):
        acc = None
        for x in range(3):
            k = x * 2 + ch
            mean = (cs_refs[k][...][0, 0] + cs_refs[6 + k][...][0, 0]) / cnt_v
            sq = mean * mean
            acc = sq if acc is None else acc + sq
        norm_parts.append(acc)
    norm = jnp.concatenate(norm_parts, axis=1)
    qc = _silu(norm @ pw1_t[...] + pb1[...])
    qcomb = _silu(qc @ pw2_t[...] + pb2[...])
    h = _silu(q_v @ wq_t[...] + agg @ wa_t[...] + qcomb @ wc_t[...]
              + nb1[...])
    out[...] = q_v + _silu(h @ nw2_t[...] + nb2[...])


def _full_spec(shape):
    return pl.BlockSpec(shape, lambda i: (0,) * len(shape))


def _row_spec(blk, width):
    return pl.BlockSpec((blk, width), lambda i: (i, 0))


def kernel(q, mu, r_ij, d_ij, mlp_in_w, mlp_in_b, mlp_out_w1, mlp_out_b1,
           mlp_out_w2, mlp_out_b2, sem_w, sem_b, xmix_w, post_w1, post_b1,
           post_w2, post_b2, node_w1, node_b1, node_w2, node_b2, idx_i, idx_j):
    n_atoms = q.shape[0]
    n_pairs = idx_i.shape[0]
    f32 = jnp.float32

    pad_rbf = lambda a, axis: jnp.concatenate(
        [a, jnp.zeros(a.shape[:axis] + (RBF_PAD - N_RBF,) + a.shape[axis + 1:],
                      f32)], axis=axis)
    wa_t = pad_rbf(mlp_in_w[:, :F].T, 1)
    wb_t = pad_rbf(mlp_in_w[:, F:].T, 1)
    b_in = pad_rbf(mlp_in_b[None, :], 1)
    offsets = jnp.linspace(0.0, CUTOFF, N_RBF)
    offs = pad_rbf(offsets[None, :], 1)
    width = offsets[1] - offsets[0]
    coeff_s = jnp.full((1, 1), -0.5 / (width ** 2), f32)
    w1i_t = mlp_out_w1[:, :F].T
    w1j_t = mlp_out_w1[:, F:2 * F].T
    w1f_t = pad_rbf(mlp_out_w1[:, 2 * F:2 * F + N_RBF], 1).T
    w1d = mlp_out_w1[:, 2 * F + N_RBF][None, :]
    b1 = mlp_out_b1[None, :]
    w2_t = mlp_out_w2.T
    b2 = mlp_out_b2[None, :]
    sem_t = sem_w.T
    semb = sem_b[None, :]
    xm = xmix_w.reshape(C, F, H)
    x0_t = xm[:, :, 0].T
    x1_t = xm[:, :, 1].T
    pw1_t = post_w1.T
    pb1 = post_b1[None, :]
    pw2_t = post_w2.T
    pb2 = post_b2[None, :]
    nq_t = node_w1[:, :F].T
    na_perm = node_w1[:, F:F + C].reshape(F, F, H).transpose(2, 1, 0)
    na_t = na_perm.reshape(C, F)
    nc_t = node_w1[:, F + C:].T
    nb1 = node_b1[None, :]
    nw2_t = node_w2.T
    nb2 = node_b2[None, :]

    pad_e = E_P - n_pairs
    idx_i_p = jnp.concatenate([idx_i, jnp.zeros((pad_e,), jnp.int32)])
    idx_j_p = jnp.concatenate([idx_j, jnp.zeros((pad_e,), jnp.int32)])
    idxcat2d = jnp.concatenate([idx_i_p, idx_j_p]).reshape(2 * E_P // 128, 128)
    idxj2d = idx_j_p.reshape(E_P // 128, 128)
    d2 = d_ij.astype(f32)
    d_p = jnp.concatenate([d2, jnp.ones((pad_e, 1), f32)], axis=0)
    r_p = jnp.concatenate([r_ij, jnp.zeros((pad_e, 3), f32)], axis=0)
    zeros128 = jnp.zeros((N_T, 128), f32)

    grid_e = E_P // E_BLK

    if not _INTERPRET:
        gathered = _sc_gather(q, idxcat2d, 2 * E_P, F)
    else:
        gathered = jnp.concatenate([q[idx_i_p], q[idx_j_p]], axis=0)

    qi_spec = pl.BlockSpec((E_BLK, F), lambda i: (i, 0))
    qj_spec = pl.BlockSpec((E_BLK, F), lambda i: (i + grid_e, 0))
    m, e128 = pl.pallas_call(
        functools.partial(_edge1_body, n_pairs),
        grid=(grid_e,),
        in_specs=[
            qi_spec, qj_spec, _row_spec(E_BLK, 1),
            _full_spec((F, RBF_PAD)), _full_spec((F, RBF_PAD)),
            _full_spec((1, RBF_PAD)), _full_spec((1, RBF_PAD)),
            _full_spec((1, 1)),
            _full_spec((F, F)), _full_spec((F, F)), _full_spec((RBF_PAD, F)),
            _full_spec((1, F)), _full_spec((1, F)), _full_spec((F, F)),
            _full_spec((1, F)), _full_spec((F, H)), _full_spec((1, H)),
        ],
        out_specs=[_row_spec(E_BLK, F), _row_spec(E_BLK, 128)],
        out_shape=[
            jax.ShapeDtypeStruct((E_P, F), f32),
            jax.ShapeDtypeStruct((E_P, 128), f32),
        ],
        interpret=_INTERPRET,
    )(gathered, gathered, d_p, wa_t, wb_t, b_in, offs, coeff_s, w1i_t, w1j_t,
      w1f_t, w1d, b1, w2_t, b2, sem_t, semb)

    if not _INTERPRET:
        parts = _sc_scatter_seg(e128, idx_j_p, zeros128)
        seg128 = parts[0] + parts[1]
        ssg128 = _sc_gather(seg128, idxj2d, E_P, 128)
    else:
        seg128 = jax.ops.segment_sum(e128, idx_j_p, num_segments=N_T)
        ssg128 = seg128[idx_j_p]

    e2_outs = pl.pallas_call(
        _edge2_body,
        grid=(grid_e,),
        in_specs=[
            _row_spec(E_BLK, F), _row_spec(E_BLK, 128), _row_spec(E_BLK, 128),
            _row_spec(E_BLK, 3), _row_spec(E_BLK, 1),
            _full_spec((F, C)), _full_spec((F, C)),
        ],
        out_specs=([_row_spec(E_BLK, C)] + [_row_spec(E_BLK, F)] * 2
                   + [_row_spec(E_BLK, 1)] * 3),
        out_shape=([jax.ShapeDtypeStruct((E_P, C), f32)]
                   + [jax.ShapeDtypeStruct((E_P, F), f32)] * 2
                   + [jax.ShapeDtypeStruct((E_P, 1), f32)] * 3),
        interpret=_INTERPRET,
    )(m, e128, ssg128, r_p, d_p, x0_t, x1_t)
    acat = e2_outs[0]
    coeff2 = tuple(e2_outs[1:3])
    rhat3 = tuple(a.reshape(E_P) for a in e2_outs[3:6])

    if not _INTERPRET:
        aggp = _sc_scatter_acat(acat, idx_j_p, zeros128)
        csp = _sc_scatter_combos(coeff2, rhat3, idx_j_p,
                                 zeros128)
    else:
        coeff = jnp.concatenate(coeff2, axis=1)
        rhat = jnp.stack(rhat3, axis=1)
        combos = jnp.stack(
            [coeff[:, ch * F:(ch + 1) * F] * rhat[:, x:x + 1]
             for x in range(3) for ch in range(2)], axis=0)
        agg1 = jax.ops.segment_sum(acat, idx_j_p, num_segments=N_T)
        cs1 = jnp.stack([jax.ops.segment_sum(combos[k], idx_j_p,
                                             num_segments=N_T)
                         for k in range(6)], axis=0)
        aggp = jnp.stack([agg1.reshape(N_T, 2, 128).transpose(1, 0, 2),
                          jnp.zeros((2, N_T, 128), f32)])
        csp = jnp.stack([cs1, jnp.zeros_like(cs1)])

    agg_specs = [
        pl.BlockSpec((1, 1, N_BLK, 128),
                     lambda i, c=c, ch=ch: (c, ch, i, 0))
        for c in range(2) for ch in range(2)
    ]
    cs_specs = [
        pl.BlockSpec((1, 1, N_BLK, 128), lambda i, c=c, k=k: (c, k, i, 0))
        for c in range(2) for k in range(6)
    ]

    grid_n = n_atoms // N_BLK
    out = pl.pallas_call(
        _node_body,
        grid=(grid_n,),
        in_specs=[_row_spec(N_BLK, F)] + agg_specs + cs_specs + [
            _row_spec(N_BLK, 128),
            _full_spec((C, F)), _full_spec((1, F)), _full_spec((F, F)),
            _full_spec((1, F)), _full_spec((F, F)), _full_spec((C, F)),
            _full_spec((F, F)), _full_spec((1, F)), _full_spec((F, F)),
            _full_spec((1, F)),
        ],
        out_specs=[_row_spec(N_BLK, F)],
        out_shape=[jax.ShapeDtypeStruct((n_atoms, F), f32)],
        interpret=_INTERPRET,
    )(q, aggp, aggp, aggp, aggp, *([csp] * 12), seg128, pw1_t, pb1, pw2_t,
      pb2, nq_t, na_t, nc_t, nb1, nw2_t, nb2)[0]
    return out


# speedup vs baseline: 6.9658x; 1.1588x over previous
import functools

import jax
import jax.numpy as jnp
from jax import lax
from jax.experimental import pallas as pl
from jax.experimental.pallas import tpu as pltpu
from jax.experimental.pallas import tpu_sc as plsc

N_ATOMS = 10000
N_PAIRS = 160000
F = 128
H = 2
C = H * F
N_RBF = 50
RBF_PAD = 64
CUTOFF = 5.0

E_BLK = 2048
N_BLK = 1000
E_P = 163840
CHUNK = 1024
N_T = 10112
ROWS_T = N_T // 16

_INTERPRET = False


def _silu(x):
    return x * jax.nn.sigmoid(x)


def _celu2(x):
    return jnp.where(x > 0, x, 2.0 * (jnp.exp(x * 0.5) - 1.0))


def _mesh():
    return plsc.VectorSubcoreMesh(core_axis_name="c", subcore_axis_name="s")


_GDN = lax.GatherDimensionNumbers(
    offset_dims=(), collapsed_slice_dims=(0,), start_index_map=(0,))


def _sc_gather(table, idx2d, B, D):
    per_w = B // 32
    iters = per_w // CHUNK

    def body(table_h, idx_h, out_h, idx_v, rows_v, sem):
        cid = lax.axis_index("c")
        sid = lax.axis_index("s")
        wid = sid * 2 + cid

        def step(i, carry):
            off = wid * per_w + i * CHUNK
            r0 = wid * (per_w // 128) + i * 8
            pltpu.sync_copy(idx_h.at[pl.ds(r0, 8)], idx_v)
            for qq in range(4):
                descs = [
                    pltpu.async_copy(table_h.at[idx_v.at[qq * 2 + s]],
                                     rows_v.at[pl.ds(s * 128, 128)], sem)
                    for s in range(2)
                ]
                for dsc in descs:
                    dsc.wait()
                pltpu.sync_copy(rows_v, out_h.at[pl.ds(off + qq * 256, 256)])
            return carry

        lax.fori_loop(0, iters, step, 0)

    return pl.kernel(
        body,
        out_type=jax.ShapeDtypeStruct((B, D), jnp.float32),
        mesh=_mesh(),
        scratch_types=[
            pltpu.VMEM((8, 128), jnp.int32),
            pltpu.VMEM((256, D), jnp.float32),
            pltpu.SemaphoreType.DMA,
        ],
    )(table, idx2d)


def _sc_scatter_seg(vals, idx1d, zeros128):
    per_sc = E_P // 2
    per_t = per_sc // 16
    iters = per_t // 128

    def body(vals_h, idx_h, z_h, out_h, idx_v, rows_v, table):
        cid = lax.axis_index("c")
        sid = lax.axis_index("s")

        @pl.when(sid == 0)
        def _():
            pltpu.sync_copy(z_h, table)

        plsc.subcore_barrier()

        def step(i, carry):
            off = cid * per_sc + sid * per_t + i * 128
            pltpu.sync_copy(vals_h.at[pl.ds(off, 128)], rows_v)
            pltpu.sync_copy(idx_h.at[pl.ds(off, 128)], idx_v)
            pltpu.sync_copy(rows_v, table.at[idx_v], add=True)
            return carry

        lax.fori_loop(0, iters, step, 0)
        plsc.subcore_barrier()

        @pl.when(sid == 0)
        def _():
            pltpu.sync_copy(table, out_h.at[cid])

    return pl.kernel(
        body,
        out_type=jax.ShapeDtypeStruct((2, N_T, 128), jnp.float32),
        mesh=_mesh(),
        scratch_types=[
            pltpu.VMEM((128,), jnp.int32),
            pltpu.VMEM((128, 128), jnp.float32),
            pltpu.VMEM_SHARED((N_T, 128), jnp.float32),
        ],
    )(vals, idx1d, zeros128)


def _sc_scatter_acat(vals, idx1d, zeros128):
    per_sc = E_P // 2
    per_t = per_sc // 16
    iters = per_t // 128

    def body(vals_h, idx_h, z_h, out_h, idx_v, rows_v, table):
        cid = lax.axis_index("c")
        sid = lax.axis_index("s")
        for ch in range(2):
            @pl.when(sid == 0)
            def _():
                pltpu.sync_copy(z_h, table)

            plsc.subcore_barrier()

            def step(i, carry):
                off = cid * per_sc + sid * per_t + i * 128
                pltpu.sync_copy(vals_h.at[pl.ds(off, 128),
                                          pl.ds(ch * 128, 128)], rows_v)
                pltpu.sync_copy(idx_h.at[pl.ds(off, 128)], idx_v)
                pltpu.sync_copy(rows_v, table.at[idx_v], add=True)
                return carry

            lax.fori_loop(0, iters, step, 0)
            plsc.subcore_barrier()

            @pl.when(sid == 0)
            def _():
                pltpu.sync_copy(table, out_h.at[cid, ch])

            plsc.subcore_barrier()

    return pl.kernel(
        body,
        out_type=jax.ShapeDtypeStruct((2, 2, N_T, 128), jnp.float32),
        mesh=_mesh(),
        scratch_types=[
            pltpu.VMEM((128,), jnp.int32),
            pltpu.VMEM((128, 128), jnp.float32),
            pltpu.VMEM_SHARED((N_T, 128), jnp.float32),
        ],
    )(vals, idx1d, zeros128)


def _sc_scatter_combos(coeff2, rhat3, idx1d, zeros128):
    per_sc = E_P // 2
    per_t = per_sc // 16
    iters = per_t // 128

    def body(c0_h, c1_h, r0_h, r1_h, r2_h, idx_h, z_h, out_h, idx_a, idx_b,
             rows_a, rows_b, srows_v, rfac_a, rfac_b, sem_a, sem_b, table):
        cid = lax.axis_index("c")
        sid = lax.axis_index("s")
        coeff_hs = [c0_h, c1_h]
        rhat_hs = [r0_h, r1_h, r2_h]
        base = cid * per_sc + sid * per_t
        last = base + per_t - 128

        def fire(ch, x, off, idx_v, rows_v, rfac_v, sem):
            o2 = jnp.minimum(off, last)
            pltpu.async_copy(coeff_hs[ch].at[pl.ds(o2, 128)], rows_v, sem)
            pltpu.async_copy(idx_h.at[pl.ds(o2, 128)], idx_v, sem)
            pltpu.async_copy(rhat_hs[x].at[pl.ds(o2, 128)], rfac_v, sem)

        def drain(ch, x, off, idx_v, rows_v, rfac_v, sem):
            o2 = jnp.minimum(off, last)
            pltpu.make_async_copy(coeff_hs[ch].at[pl.ds(o2, 128)], rows_v,
                                  sem).wait()
            pltpu.make_async_copy(idx_h.at[pl.ds(o2, 128)], idx_v,
                                  sem).wait()
            pltpu.make_async_copy(rhat_hs[x].at[pl.ds(o2, 128)], rfac_v,
                                  sem).wait()

        def scale_scatter(idx_v, rows_v, rfac_v):
            def gstep(g, cc):
                base16 = (g // 2) * 16
                half = (g % 2) * 8
                rx = rfac_v[pl.ds(base16, 16)]
                for j in range(8):
                    row = base16 + half + j
                    sidx = jnp.zeros((16, 1), jnp.int32) + (half + j)
                    sc = lax.gather(
                        rx, sidx, _GDN, (1,),
                        mode=lax.GatherScatterMode.PROMISE_IN_BOUNDS)
                    for v in range(8):
                        srows_v[row, pl.ds(v * 16, 16)] = (
                            rows_v[row, pl.ds(v * 16, 16)] * sc)
                return cc

            lax.fori_loop(0, 16, gstep, 0)
            pltpu.sync_copy(srows_v, table.at[idx_v], add=True)

        for x in range(3):
            for ch in range(2):
                @pl.when(sid == 0)
                def _():
                    pltpu.sync_copy(z_h, table)

                plsc.subcore_barrier()
                fire(ch, x, base, idx_a, rows_a, rfac_a, sem_a)

                def step(i2, carry):
                    off_a = base + i2 * 256
                    fire(ch, x, off_a + 128, idx_b, rows_b, rfac_b, sem_b)
                    drain(ch, x, off_a, idx_a, rows_a, rfac_a, sem_a)
                    scale_scatter(idx_a, rows_a, rfac_a)
                    fire(ch, x, off_a + 256, idx_a, rows_a, rfac_a, sem_a)
                    drain(ch, x, off_a + 128, idx_b, rows_b, rfac_b, sem_b)
                    scale_scatter(idx_b, rows_b, rfac_b)
                    return carry

                lax.fori_loop(0, iters // 2, step, 0)
                drain(ch, x, base + per_t, idx_a, rows_a, rfac_a, sem_a)
                plsc.subcore_barrier()

                @pl.when(sid == 0)
                def _():
                    pltpu.sync_copy(table, out_h.at[cid, x * 2 + ch])

                plsc.subcore_barrier()

    return pl.kernel(
        body,
        out_type=jax.ShapeDtypeStruct((2, 6, N_T, 128), jnp.float32),
        mesh=_mesh(),
        compiler_params=pltpu.CompilerParams(needs_layout_passes=False),
        scratch_types=[
            pltpu.VMEM((128,), jnp.int32),
            pltpu.VMEM((128,), jnp.int32),
            pltpu.VMEM((128, 128), jnp.float32),
            pltpu.VMEM((128, 128), jnp.float32),
            pltpu.VMEM((128, 128), jnp.float32),
            pltpu.VMEM((128,), jnp.float32),
            pltpu.VMEM((128,), jnp.float32),
            pltpu.SemaphoreType.DMA,
            pltpu.SemaphoreType.DMA,
            pltpu.VMEM_SHARED((N_T, 128), jnp.float32),
        ],
    )(*coeff2, *rhat3, idx1d, zeros128)


def _edge1_body(n_valid, qi, qj, d, wa_t, wb_t, b_in, offs, coeff_s, w1i_t,
                w1j_t, w1f_t, w1d, b1, w2_t, b2, sem_t, semb, m_out, e_out):
    qi_v = qi[...]
    qj_v = qj[...]
    d_v = d[...]
    qe = qi_v @ wa_t[...] + qj_v @ wb_t[...] + b_in[...]
    rbf = jnp.exp(coeff_s[0, 0] * (d_v - offs[...]) ** 2)
    filt = rbf * qe
    pre = (qi_v @ w1i_t[...] + qj_v @ w1j_t[...] + filt @ w1f_t[...]
           + d_v * w1d[...] + b1[...])
    h = _silu(pre)
    m = h @ w2_t[...] + b2[...]
    att = _celu2(m @ sem_t[...] + semb[...])
    e = jnp.exp(jnp.minimum(att, 60.0))
    rows = (pl.program_id(0) * E_BLK
            + lax.broadcasted_iota(jnp.int32, (E_BLK, 1), 0))
    valid = rows < n_valid
    e128 = jnp.concatenate(
        [e, jnp.ones((E_BLK, 1), jnp.float32),
         jnp.zeros((E_BLK, 125), jnp.float32)], axis=1)
    m_out[...] = m
    e_out[...] = jnp.where(valid, e128, 0.0)


def _edge2_body(m, e128, ssg128, r, d, x0_t, x1_t, acat_out, c0_out, c1_out,
                r0_out, r1_out, r2_out):
    m_v = m[...]
    e = e128[...][:, 0:H]
    ssg = jnp.maximum(ssg128[...][:, 0:H], 1e-30)
    comb = e / ssg
    a0 = m_v * comb[:, 0:1]
    a1 = m_v * comb[:, 1:2]
    coeff = jnp.tanh(a0 @ x0_t[...] + a1 @ x1_t[...])
    acat_out[...] = jnp.concatenate([a0, a1], axis=1)
    c0_out[...] = coeff[:, 0:F]
    c1_out[...] = coeff[:, F:C]
    rhat = r[...] / (d[...] + 1e-05)
    r0_out[...] = rhat[:, 0:1]
    r1_out[...] = rhat[:, 1:2]
    r2_out[...] = rhat[:, 2:3]


def _node_body(*refs):
    (q, agg0a, agg0b, agg1a, agg1b), cs_refs = refs[0:5], refs[5:17]
    (cnt128, pw1_t, pb1, pw2_t, pb2, wq_t, wa_t, wc_t, nb1, nw2_t, nb2,
     out) = refs[17:]
    q_v = q[...]
    agg = jnp.concatenate(
        [agg0a[...][0, 0] + agg1a[...][0, 0],
         agg0b[...][0, 0] + agg1b[...][0, 0]], axis=1)
    cnt_v = jnp.maximum(cnt128[...][:, 2:3], 1.0)
    norm_parts = []
    for ch in range(2):
        acc = None
        for x in range(3):
            k = x * 2 + ch
            mean = (cs_refs[k][...][0, 0] + cs_refs[6 + k][...][0, 0]) / cnt_v
            sq = mean * mean
            acc = sq if acc is None else acc + sq
        norm_parts.append(acc)
    norm = jnp.concatenate(norm_parts, axis=1)
    qc = _silu(norm @ pw1_t[...] + pb1[...])
    qcomb = _silu(qc @ pw2_t[...] + pb2[...])
    h = _silu(q_v @ wq_t[...] + agg @ wa_t[...] + qcomb @ wc_t[...]
              + nb1[...])
    out[...] = q_v + _silu(h @ nw2_t[...] + nb2[...])


def _full_spec(shape):
    return pl.BlockSpec(shape, lambda i: (0,) * len(shape))


def _row_spec(blk, width):
    return pl.BlockSpec((blk, width), lambda i: (i, 0))


def kernel(q, mu, r_ij, d_ij, mlp_in_w, mlp_in_b, mlp_out_w1, mlp_out_b1,
           mlp_out_w2, mlp_out_b2, sem_w, sem_b, xmix_w, post_w1, post_b1,
           post_w2, post_b2, node_w1, node_b1, node_w2, node_b2, idx_i, idx_j):
    n_atoms = q.shape[0]
    n_pairs = idx_i.shape[0]
    f32 = jnp.float32

    pad_rbf = lambda a, axis: jnp.concatenate(
        [a, jnp.zeros(a.shape[:axis] + (RBF_PAD - N_RBF,) + a.shape[axis + 1:],
                      f32)], axis=axis)
    wa_t = pad_rbf(mlp_in_w[:, :F].T, 1)
    wb_t = pad_rbf(mlp_in_w[:, F:].T, 1)
    b_in = pad_rbf(mlp_in_b[None, :], 1)
    offsets = jnp.linspace(0.0, CUTOFF, N_RBF)
    offs = pad_rbf(offsets[None, :], 1)
    width = offsets[1] - offsets[0]
    coeff_s = jnp.full((1, 1), -0.5 / (width ** 2), f32)
    w1i_t = mlp_out_w1[:, :F].T
    w1j_t = mlp_out_w1[:, F:2 * F].T
    w1f_t = pad_rbf(mlp_out_w1[:, 2 * F:2 * F + N_RBF], 1).T
    w1d = mlp_out_w1[:, 2 * F + N_RBF][None, :]
    b1 = mlp_out_b1[None, :]
    w2_t = mlp_out_w2.T
    b2 = mlp_out_b2[None, :]
    sem_t = sem_w.T
    semb = sem_b[None, :]
    xm = xmix_w.reshape(C, F, H)
    x0_t = xm[:, :, 0].T
    x1_t = xm[:, :, 1].T
    pw1_t = post_w1.T
    pb1 = post_b1[None, :]
    pw2_t = post_w2.T
    pb2 = post_b2[None, :]
    nq_t = node_w1[:, :F].T
    na_perm = node_w1[:, F:F + C].reshape(F, F, H).transpose(2, 1, 0)
    na_t = na_perm.reshape(C, F)
    nc_t = node_w1[:, F + C:].T
    nb1 = node_b1[None, :]
    nw2_t = node_w2.T
    nb2 = node_b2[None, :]

    pad_e = E_P - n_pairs
    idx_i_p = jnp.concatenate([idx_i, jnp.zeros((pad_e,), jnp.int32)])
    idx_j_p = jnp.concatenate([idx_j, jnp.zeros((pad_e,), jnp.int32)])
    idxcat2d = jnp.concatenate([idx_i_p, idx_j_p]).reshape(2 * E_P // 128, 128)
    idxj2d = idx_j_p.reshape(E_P // 128, 128)
    d2 = d_ij.astype(f32)
    d_p = jnp.concatenate([d2, jnp.ones((pad_e, 1), f32)], axis=0)
    r_p = jnp.concatenate([r_ij, jnp.zeros((pad_e, 3), f32)], axis=0)
    zeros128 = jnp.zeros((N_T, 128), f32)

    grid_e = E_P // E_BLK

    if not _INTERPRET:
        gathered = _sc_gather(q, idxcat2d, 2 * E_P, F)
    else:
        gathered = jnp.concatenate([q[idx_i_p], q[idx_j_p]], axis=0)

    qi_spec = pl.BlockSpec((E_BLK, F), lambda i: (i, 0))
    qj_spec = pl.BlockSpec((E_BLK, F), lambda i: (i + grid_e, 0))
    m, e128 = pl.pallas_call(
        functools.partial(_edge1_body, n_pairs),
        grid=(grid_e,),
        in_specs=[
            qi_spec, qj_spec, _row_spec(E_BLK, 1),
            _full_spec((F, RBF_PAD)), _full_spec((F, RBF_PAD)),
            _full_spec((1, RBF_PAD)), _full_spec((1, RBF_PAD)),
            _full_spec((1, 1)),
            _full_spec((F, F)), _full_spec((F, F)), _full_spec((RBF_PAD, F)),
            _full_spec((1, F)), _full_spec((1, F)), _full_spec((F, F)),
            _full_spec((1, F)), _full_spec((F, H)), _full_spec((1, H)),
        ],
        out_specs=[_row_spec(E_BLK, F), _row_spec(E_BLK, 128)],
        out_shape=[
            jax.ShapeDtypeStruct((E_P, F), f32),
            jax.ShapeDtypeStruct((E_P, 128), f32),
        ],
        interpret=_INTERPRET,
    )(gathered, gathered, d_p, wa_t, wb_t, b_in, offs, coeff_s, w1i_t, w1j_t,
      w1f_t, w1d, b1, w2_t, b2, sem_t, semb)

    if not _INTERPRET:
        parts = _sc_scatter_seg(e128, idx_j_p, zeros128)
        seg128 = parts[0] + parts[1]
        ssg128 = _sc_gather(seg128, idxj2d, E_P, 128)
    else:
        seg128 = jax.ops.segment_sum(e128, idx_j_p, num_segments=N_T)
        ssg128 = seg128[idx_j_p]

    e2_outs = pl.pallas_call(
        _edge2_body,
        grid=(grid_e,),
        in_specs=[
            _row_spec(E_BLK, F), _row_spec(E_BLK, 128), _row_spec(E_BLK, 128),
            _row_spec(E_BLK, 3), _row_spec(E_BLK, 1),
            _full_spec((F, C)), _full_spec((F, C)),
        ],
        out_specs=([_row_spec(E_BLK, C)] + [_row_spec(E_BLK, F)] * 2
                   + [_row_spec(E_BLK, 1)] * 3),
        out_shape=([jax.ShapeDtypeStruct((E_P, C), f32)]
                   + [jax.ShapeDtypeStruct((E_P, F), f32)] * 2
                   + [jax.ShapeDtypeStruct((E_P, 1), f32)] * 3),
        interpret=_INTERPRET,
    )(m, e128, ssg128, r_p, d_p, x0_t, x1_t)
    acat = e2_outs[0]
    coeff2 = tuple(e2_outs[1:3])
    rhat3 = tuple(a.reshape(E_P) for a in e2_outs[3:6])

    if not _INTERPRET:
        aggp = _sc_scatter_acat(acat, idx_j_p, zeros128)
        csp = _sc_scatter_combos(coeff2, rhat3, idx_j_p,
                                 zeros128)
    else:
        coeff = jnp.concatenate(coeff2, axis=1)
        rhat = jnp.stack(rhat3, axis=1)
        combos = jnp.stack(
            [coeff[:, ch * F:(ch + 1) * F] * rhat[:, x:x + 1]
             for x in range(3) for ch in range(2)], axis=0)
        agg1 = jax.ops.segment_sum(acat, idx_j_p, num_segments=N_T)
        cs1 = jnp.stack([jax.ops.segment_sum(combos[k], idx_j_p,
                                             num_segments=N_T)
                         for k in range(6)], axis=0)
        aggp = jnp.stack([agg1.reshape(N_T, 2, 128).transpose(1, 0, 2),
                          jnp.zeros((2, N_T, 128), f32)])
        csp = jnp.stack([cs1, jnp.zeros_like(cs1)])

    agg_specs = [
        pl.BlockSpec((1, 1, N_BLK, 128),
                     lambda i, c=c, ch=ch: (c, ch, i, 0))
        for c in range(2) for ch in range(2)
    ]
    cs_specs = [
        pl.BlockSpec((1, 1, N_BLK, 128), lambda i, c=c, k=k: (c, k, i, 0))
        for c in range(2) for k in range(6)
    ]

    grid_n = n_atoms // N_BLK
    out = pl.pallas_call(
        _node_body,
        grid=(grid_n,),
        in_specs=[_row_spec(N_BLK, F)] + agg_specs + cs_specs + [
            _row_spec(N_BLK, 128),
            _full_spec((C, F)), _full_spec((1, F)), _full_spec((F, F)),
            _full_spec((1, F)), _full_spec((F, F)), _full_spec((C, F)),
            _full_spec((F, F)), _full_spec((1, F)), _full_spec((F, F)),
            _full_spec((1, F)),
        ],
        out_specs=[_row_spec(N_BLK, F)],
        out_shape=[jax.ShapeDtypeStruct((n_atoms, F), f32)],
        interpret=_INTERPRET,
    )(q, aggp, aggp, aggp, aggp, *([csp] * 12), seg128, pw1_t, pb1, pw2_t,
      pb2, nq_t, na_t, nc_t, nb1, nw2_t, nb2)[0]
    return out


# speedup vs baseline: 7.3119x; 1.0497x over previous
import functools

import jax
import jax.numpy as jnp
from jax import lax
from jax.experimental import pallas as pl
from jax.experimental.pallas import tpu as pltpu
from jax.experimental.pallas import tpu_sc as plsc

N_ATOMS = 10000
N_PAIRS = 160000
F = 128
H = 2
C = H * F
N_RBF = 50
RBF_PAD = 64
CUTOFF = 5.0

E_BLK = 2048
N_BLK = 1000
E_P = 163840
CHUNK = 1024
N_T = 10112
ROWS_T = N_T // 16

_INTERPRET = False


def _silu(x):
    return x * jax.nn.sigmoid(x)


def _celu2(x):
    return jnp.where(x > 0, x, 2.0 * (jnp.exp(x * 0.5) - 1.0))


def _mesh():
    return plsc.VectorSubcoreMesh(core_axis_name="c", subcore_axis_name="s")


_GDN = lax.GatherDimensionNumbers(
    offset_dims=(), collapsed_slice_dims=(0,), start_index_map=(0,))


def _sc_gather(table, idx2d, B, D):
    per_w = B // 32
    iters = per_w // CHUNK

    def body(table_h, idx_h, out_h, idx_v, rows_v, sem):
        cid = lax.axis_index("c")
        sid = lax.axis_index("s")
        wid = sid * 2 + cid

        def step(i, carry):
            off = wid * per_w + i * CHUNK
            r0 = wid * (per_w // 128) + i * 8
            pltpu.sync_copy(idx_h.at[pl.ds(r0, 8)], idx_v)
            for qq in range(4):
                descs = [
                    pltpu.async_copy(table_h.at[idx_v.at[qq * 2 + s]],
                                     rows_v.at[pl.ds(s * 128, 128)], sem)
                    for s in range(2)
                ]
                for dsc in descs:
                    dsc.wait()
                pltpu.sync_copy(rows_v, out_h.at[pl.ds(off + qq * 256, 256)])
            return carry

        lax.fori_loop(0, iters, step, 0)

    return pl.kernel(
        body,
        out_type=jax.ShapeDtypeStruct((B, D), jnp.float32),
        mesh=_mesh(),
        scratch_types=[
            pltpu.VMEM((8, 128), jnp.int32),
            pltpu.VMEM((256, D), jnp.float32),
            pltpu.SemaphoreType.DMA,
        ],
    )(table, idx2d)


def _sc_scatter_seg(vals, idx1d, zeros128):
    per_sc = E_P // 2
    per_t = per_sc // 16
    iters = per_t // 128

    def body(vals_h, idx_h, z_h, out_h, idx_a, idx_b, rows_a, rows_b,
             sem_a, sem_b, table):
        cid = lax.axis_index("c")
        sid = lax.axis_index("s")
        base = cid * per_sc + sid * per_t
        last = base + per_t - 128

        def fire(off, idx_v, rows_v, sem):
            o2 = jnp.minimum(off, last)
            pltpu.async_copy(vals_h.at[pl.ds(o2, 128)], rows_v, sem)
            pltpu.async_copy(idx_h.at[pl.ds(o2, 128)], idx_v, sem)

        def drain(off, idx_v, rows_v, sem):
            o2 = jnp.minimum(off, last)
            pltpu.make_async_copy(vals_h.at[pl.ds(o2, 128)], rows_v,
                                  sem).wait()
            pltpu.make_async_copy(idx_h.at[pl.ds(o2, 128)], idx_v,
                                  sem).wait()

        @pl.when(sid == 0)
        def _():
            pltpu.sync_copy(z_h, table)

        plsc.subcore_barrier()
        fire(base, idx_a, rows_a, sem_a)

        def step(i2, carry):
            off_a = base + i2 * 256
            fire(off_a + 128, idx_b, rows_b, sem_b)
            drain(off_a, idx_a, rows_a, sem_a)
            pltpu.sync_copy(rows_a, table.at[idx_a], add=True)
            fire(off_a + 256, idx_a, rows_a, sem_a)
            drain(off_a + 128, idx_b, rows_b, sem_b)
            pltpu.sync_copy(rows_b, table.at[idx_b], add=True)
            return carry

        lax.fori_loop(0, iters // 2, step, 0)
        drain(base + per_t, idx_a, rows_a, sem_a)
        plsc.subcore_barrier()

        @pl.when(sid == 0)
        def _():
            pltpu.sync_copy(table, out_h.at[cid])

    return pl.kernel(
        body,
        out_type=jax.ShapeDtypeStruct((2, N_T, 128), jnp.float32),
        mesh=_mesh(),
        scratch_types=[
            pltpu.VMEM((128,), jnp.int32),
            pltpu.VMEM((128,), jnp.int32),
            pltpu.VMEM((128, 128), jnp.float32),
            pltpu.VMEM((128, 128), jnp.float32),
            pltpu.SemaphoreType.DMA,
            pltpu.SemaphoreType.DMA,
            pltpu.VMEM_SHARED((N_T, 128), jnp.float32),
        ],
    )(vals, idx1d, zeros128)


def _sc_scatter_acat(vals, idx1d, zeros128):
    per_sc = E_P // 2
    per_t = per_sc // 16
    iters = per_t // 128

    def body(vals_h, idx_h, z_h, out_h, idx_a, idx_b, rows_a, rows_b,
             sem_a, sem_b, table):
        cid = lax.axis_index("c")
        sid = lax.axis_index("s")
        base = cid * per_sc + sid * per_t
        last = base + per_t - 128

        def fire(ch, off, idx_v, rows_v, sem):
            o2 = jnp.minimum(off, last)
            pltpu.async_copy(vals_h.at[pl.ds(o2, 128), pl.ds(ch * 128, 128)],
                             rows_v, sem)
            pltpu.async_copy(idx_h.at[pl.ds(o2, 128)], idx_v, sem)

        def drain(ch, off, idx_v, rows_v, sem):
            o2 = jnp.minimum(off, last)
            pltpu.make_async_copy(
                vals_h.at[pl.ds(o2, 128), pl.ds(ch * 128, 128)], rows_v,
                sem).wait()
            pltpu.make_async_copy(idx_h.at[pl.ds(o2, 128)], idx_v,
                                  sem).wait()

        for ch in range(2):
            @pl.when(sid == 0)
            def _():
                pltpu.sync_copy(z_h, table)

            plsc.subcore_barrier()
            fire(ch, base, idx_a, rows_a, sem_a)

            def step(i2, carry):
                off_a = base + i2 * 256
                fire(ch, off_a + 128, idx_b, rows_b, sem_b)
                drain(ch, off_a, idx_a, rows_a, sem_a)
                pltpu.sync_copy(rows_a, table.at[idx_a], add=True)
                fire(ch, off_a + 256, idx_a, rows_a, sem_a)
                drain(ch, off_a + 128, idx_b, rows_b, sem_b)
                pltpu.sync_copy(rows_b, table.at[idx_b], add=True)
                return carry

            lax.fori_loop(0, iters // 2, step, 0)
            drain(ch, base + per_t, idx_a, rows_a, sem_a)
            plsc.subcore_barrier()

            @pl.when(sid == 0)
            def _():
                pltpu.sync_copy(table, out_h.at[cid, ch])

            plsc.subcore_barrier()

    return pl.kernel(
        body,
        out_type=jax.ShapeDtypeStruct((2, 2, N_T, 128), jnp.float32),
        mesh=_mesh(),
        scratch_types=[
            pltpu.VMEM((128,), jnp.int32),
            pltpu.VMEM((128,), jnp.int32),
            pltpu.VMEM((128, 128), jnp.float32),
            pltpu.VMEM((128, 128), jnp.float32),
            pltpu.SemaphoreType.DMA,
            pltpu.SemaphoreType.DMA,
            pltpu.VMEM_SHARED((N_T, 128), jnp.float32),
        ],
    )(vals, idx1d, zeros128)


def _sc_scatter_combos(coeff2, rhat3, idx1d, zeros128):
    per_sc = E_P // 2
    per_t = per_sc // 16
    iters = per_t // 128

    def body(c0_h, c1_h, r0_h, r1_h, r2_h, idx_h, z_h, out_h, idx_a, idx_b,
             rows_a, rows_b, srows_v, rfac_a, rfac_b, sem_a, sem_b, table):
        cid = lax.axis_index("c")
        sid = lax.axis_index("s")
        coeff_hs = [c0_h, c1_h]
        rhat_hs = [r0_h, r1_h, r2_h]
        base = cid * per_sc + sid * per_t
        last = base + per_t - 128

        def fire(ch, x, off, idx_v, rows_v, rfac_v, sem):
            o2 = jnp.minimum(off, last)
            pltpu.async_copy(coeff_hs[ch].at[pl.ds(o2, 128)], rows_v, sem)
            pltpu.async_copy(idx_h.at[pl.ds(o2, 128)], idx_v, sem)
            pltpu.async_copy(rhat_hs[x].at[pl.ds(o2, 128)], rfac_v, sem)

        def drain(ch, x, off, idx_v, rows_v, rfac_v, sem):
            o2 = jnp.minimum(off, last)
            pltpu.make_async_copy(coeff_hs[ch].at[pl.ds(o2, 128)], rows_v,
                                  sem).wait()
            pltpu.make_async_copy(idx_h.at[pl.ds(o2, 128)], idx_v,
                                  sem).wait()
            pltpu.make_async_copy(rhat_hs[x].at[pl.ds(o2, 128)], rfac_v,
                                  sem).wait()

        def scale_scatter(idx_v, rows_v, rfac_v):
            def gstep(g, cc):
                base16 = (g // 2) * 16
                half = (g % 2) * 8
                rx = rfac_v[pl.ds(base16, 16)]
                for j in range(8):
                    row = base16 + half + j
                    sidx = jnp.zeros((16, 1), jnp.int32) + (half + j)
                    sc = lax.gather(
                        rx, sidx, _GDN, (1,),
                        mode=lax.GatherScatterMode.PROMISE_IN_BOUNDS)
                    for v in range(8):
                        srows_v[row, pl.ds(v * 16, 16)] = (
                            rows_v[row, pl.ds(v * 16, 16)] * sc)
                return cc

            lax.fori_loop(0, 16, gstep, 0)
            pltpu.sync_copy(srows_v, table.at[idx_v], add=True)

        for x in range(3):
            for ch in range(2):
                @pl.when(sid == 0)
                def _():
                    pltpu.sync_copy(z_h, table)

                plsc.subcore_barrier()
                fire(ch, x, base, idx_a, rows_a, rfac_a, sem_a)

                def step(i2, carry):
                    off_a = base + i2 * 256
                    fire(ch, x, off_a + 128, idx_b, rows_b, rfac_b, sem_b)
                    drain(ch, x, off_a, idx_a, rows_a, rfac_a, sem_a)
                    scale_scatter(idx_a, rows_a, rfac_a)
                    fire(ch, x, off_a + 256, idx_a, rows_a, rfac_a, sem_a)
                    drain(ch, x, off_a + 128, idx_b, rows_b, rfac_b, sem_b)
                    scale_scatter(idx_b, rows_b, rfac_b)
                    return carry

                lax.fori_loop(0, iters // 2, step, 0)
                drain(ch, x, base + per_t, idx_a, rows_a, rfac_a, sem_a)
                plsc.subcore_barrier()

                @pl.when(sid == 0)
                def _():
                    pltpu.sync_copy(table, out_h.at[cid, x * 2 + ch])

                plsc.subcore_barrier()

    return pl.kernel(
        body,
        out_type=jax.ShapeDtypeStruct((2, 6, N_T, 128), jnp.float32),
        mesh=_mesh(),
        compiler_params=pltpu.CompilerParams(needs_layout_passes=False),
        scratch_types=[
            pltpu.VMEM((128,), jnp.int32),
            pltpu.VMEM((128,), jnp.int32),
            pltpu.VMEM((128, 128), jnp.float32),
            pltpu.VMEM((128, 128), jnp.float32),
            pltpu.VMEM((128, 128), jnp.float32),
            pltpu.VMEM((128,), jnp.float32),
            pltpu.VMEM((128,), jnp.float32),
            pltpu.SemaphoreType.DMA,
            pltpu.SemaphoreType.DMA,
            pltpu.VMEM_SHARED((N_T, 128), jnp.float32),
        ],
    )(*coeff2, *rhat3, idx1d, zeros128)


def _edge1_body(n_valid, qi, qj, d, wa_t, wb_t, b_in, offs, coeff_s, w1i_t,
                w1j_t, w1f_t, w1d, b1, w2_t, b2, sem_t, semb, m_out, e_out):
    qi_v = qi[...]
    qj_v = qj[...]
    d_v = d[...]
    qe = qi_v @ wa_t[...] + qj_v @ wb_t[...] + b_in[...]
    rbf = jnp.exp(coeff_s[0, 0] * (d_v - offs[...]) ** 2)
    filt = rbf * qe
    pre = (qi_v @ w1i_t[...] + qj_v @ w1j_t[...] + filt @ w1f_t[...]
           + d_v * w1d[...] + b1[...])
    h = _silu(pre)
    m = h @ w2_t[...] + b2[...]
    att = _celu2(m @ sem_t[...] + semb[...])
    e = jnp.exp(jnp.minimum(att, 60.0))
    rows = (pl.program_id(0) * E_BLK
            + lax.broadcasted_iota(jnp.int32, (E_BLK, 1), 0))
    valid = rows < n_valid
    e128 = jnp.concatenate(
        [e, jnp.ones((E_BLK, 1), jnp.float32),
         jnp.zeros((E_BLK, 125), jnp.float32)], axis=1)
    m_out[...] = m
    e_out[...] = jnp.where(valid, e128, 0.0)


def _edge2_body(m, e128, ssg128, r, d, x0_t, x1_t, acat_out, c0_out, c1_out,
                r0_out, r1_out, r2_out):
    m_v = m[...]
    e = e128[...][:, 0:H]
    ssg = jnp.maximum(ssg128[...][:, 0:H], 1e-30)
    comb = e / ssg
    a0 = m_v * comb[:, 0:1]
    a1 = m_v * comb[:, 1:2]
    coeff = jnp.tanh(a0 @ x0_t[...] + a1 @ x1_t[...])
    acat_out[...] = jnp.concatenate([a0, a1], axis=1)
    c0_out[...] = coeff[:, 0:F]
    c1_out[...] = coeff[:, F:C]
    rhat = r[...] / (d[...] + 1e-05)
    r0_out[...] = rhat[:, 0:1]
    r1_out[...] = rhat[:, 1:2]
    r2_out[...] = rhat[:, 2:3]


def _node_body(*refs):
    (q, agg0a, agg0b, agg1a, agg1b), cs_refs = refs[0:5], refs[5:17]
    (cnt128, pw1_t, pb1, pw2_t, pb2, wq_t, wa_t, wc_t, nb1, nw2_t, nb2,
     out) = refs[17:]
    q_v = q[...]
    agg = jnp.concatenate(
        [agg0a[...][0, 0] + agg1a[...][0, 0],
         agg0b[...][0, 0] + agg1b[...][0, 0]], axis=1)
    cnt_v = jnp.maximum(cnt128[...][:, 2:3], 1.0)
    norm_parts = []
    for ch in range(2):
        acc = None
        for x in range(3):
            k = x * 2 + ch
            mean = (cs_refs[k][...][0, 0] + cs_refs[6 + k][...][0, 0]) / cnt_v
            sq = mean * mean
            acc = sq if acc is None else acc + sq
        norm_parts.append(acc)
    norm = jnp.concatenate(norm_parts, axis=1)
    qc = _silu(norm @ pw1_t[...] + pb1[...])
    qcomb = _silu(qc @ pw2_t[...] + pb2[...])
    h = _silu(q_v @ wq_t[...] + agg @ wa_t[...] + qcomb @ wc_t[...]
              + nb1[...])
    out[...] = q_v + _silu(h @ nw2_t[...] + nb2[...])


def _full_spec(shape):
    return pl.BlockSpec(shape, lambda i: (0,) * len(shape))


def _row_spec(blk, width):
    return pl.BlockSpec((blk, width), lambda i: (i, 0))


def kernel(q, mu, r_ij, d_ij, mlp_in_w, mlp_in_b, mlp_out_w1, mlp_out_b1,
           mlp_out_w2, mlp_out_b2, sem_w, sem_b, xmix_w, post_w1, post_b1,
           post_w2, post_b2, node_w1, node_b1, node_w2, node_b2, idx_i, idx_j):
    n_atoms = q.shape[0]
    n_pairs = idx_i.shape[0]
    f32 = jnp.float32

    pad_rbf = lambda a, axis: jnp.concatenate(
        [a, jnp.zeros(a.shape[:axis] + (RBF_PAD - N_RBF,) + a.shape[axis + 1:],
                      f32)], axis=axis)
    wa_t = pad_rbf(mlp_in_w[:, :F].T, 1)
    wb_t = pad_rbf(mlp_in_w[:, F:].T, 1)
    b_in = pad_rbf(mlp_in_b[None, :], 1)
    offsets = jnp.linspace(0.0, CUTOFF, N_RBF)
    offs = pad_rbf(offsets[None, :], 1)
    width = offsets[1] - offsets[0]
    coeff_s = jnp.full((1, 1), -0.5 / (width ** 2), f32)
    w1i_t = mlp_out_w1[:, :F].T
    w1j_t = mlp_out_w1[:, F:2 * F].T
    w1f_t = pad_rbf(mlp_out_w1[:, 2 * F:2 * F + N_RBF], 1).T
    w1d = mlp_out_w1[:, 2 * F + N_RBF][None, :]
    b1 = mlp_out_b1[None, :]
    w2_t = mlp_out_w2.T
    b2 = mlp_out_b2[None, :]
    sem_t = sem_w.T
    semb = sem_b[None, :]
    xm = xmix_w.reshape(C, F, H)
    x0_t = xm[:, :, 0].T
    x1_t = xm[:, :, 1].T
    pw1_t = post_w1.T
    pb1 = post_b1[None, :]
    pw2_t = post_w2.T
    pb2 = post_b2[None, :]
    nq_t = node_w1[:, :F].T
    na_perm = node_w1[:, F:F + C].reshape(F, F, H).transpose(2, 1, 0)
    na_t = na_perm.reshape(C, F)
    nc_t = node_w1[:, F + C:].T
    nb1 = node_b1[None, :]
    nw2_t = node_w2.T
    nb2 = node_b2[None, :]

    pad_e = E_P - n_pairs
    idx_i_p = jnp.concatenate([idx_i, jnp.zeros((pad_e,), jnp.int32)])
    idx_j_p = jnp.concatenate([idx_j, jnp.zeros((pad_e,), jnp.int32)])
    idxcat2d = jnp.concatenate([idx_i_p, idx_j_p]).reshape(2 * E_P // 128, 128)
    idxj2d = idx_j_p.reshape(E_P // 128, 128)
    d2 = d_ij.astype(f32)
    d_p = jnp.concatenate([d2, jnp.ones((pad_e, 1), f32)], axis=0)
    r_p = jnp.concatenate([r_ij, jnp.zeros((pad_e, 3), f32)], axis=0)
    zeros128 = jnp.zeros((N_T, 128), f32)

    grid_e = E_P // E_BLK

    if not _INTERPRET:
        gathered = _sc_gather(q, idxcat2d, 2 * E_P, F)
    else:
        gathered = jnp.concatenate([q[idx_i_p], q[idx_j_p]], axis=0)

    qi_spec = pl.BlockSpec((E_BLK, F), lambda i: (i, 0))
    qj_spec = pl.BlockSpec((E_BLK, F), lambda i: (i + grid_e, 0))
    m, e128 = pl.pallas_call(
        functools.partial(_edge1_body, n_pairs),
        grid=(grid_e,),
        in_specs=[
            qi_spec, qj_spec, _row_spec(E_BLK, 1),
            _full_spec((F, RBF_PAD)), _full_spec((F, RBF_PAD)),
            _full_spec((1, RBF_PAD)), _full_spec((1, RBF_PAD)),
            _full_spec((1, 1)),
            _full_spec((F, F)), _full_spec((F, F)), _full_spec((RBF_PAD, F)),
            _full_spec((1, F)), _full_spec((1, F)), _full_spec((F, F)),
            _full_spec((1, F)), _full_spec((F, H)), _full_spec((1, H)),
        ],
        out_specs=[_row_spec(E_BLK, F), _row_spec(E_BLK, 128)],
        out_shape=[
            jax.ShapeDtypeStruct((E_P, F), f32),
            jax.ShapeDtypeStruct((E_P, 128), f32),
        ],
        interpret=_INTERPRET,
    )(gathered, gathered, d_p, wa_t, wb_t, b_in, offs, coeff_s, w1i_t, w1j_t,
      w1f_t, w1d, b1, w2_t, b2, sem_t, semb)

    if not _INTERPRET:
        parts = _sc_scatter_seg(e128, idx_j_p, zeros128)
        seg128 = parts[0] + parts[1]
        ssg128 = _sc_gather(seg128, idxj2d, E_P, 128)
    else:
        seg128 = jax.ops.segment_sum(e128, idx_j_p, num_segments=N_T)
        ssg128 = seg128[idx_j_p]

    e2_outs = pl.pallas_call(
        _edge2_body,
        grid=(grid_e,),
        in_specs=[
            _row_spec(E_BLK, F), _row_spec(E_BLK, 128), _row_spec(E_BLK, 128),
            _row_spec(E_BLK, 3), _row_spec(E_BLK, 1),
            _full_spec((F, C)), _full_spec((F, C)),
        ],
        out_specs=([_row_spec(E_BLK, C)] + [_row_spec(E_BLK, F)] * 2
                   + [_row_spec(E_BLK, 1)] * 3),
        out_shape=([jax.ShapeDtypeStruct((E_P, C), f32)]
                   + [jax.ShapeDtypeStruct((E_P, F), f32)] * 2
                   + [jax.ShapeDtypeStruct((E_P, 1), f32)] * 3),
        interpret=_INTERPRET,
    )(m, e128, ssg128, r_p, d_p, x0_t, x1_t)
    acat = e2_outs[0]
    coeff2 = tuple(e2_outs[1:3])
    rhat3 = tuple(a.reshape(E_P) for a in e2_outs[3:6])

    if not _INTERPRET:
        aggp = _sc_scatter_acat(acat, idx_j_p, zeros128)
        csp = _sc_scatter_combos(coeff2, rhat3, idx_j_p,
                                 zeros128)
    else:
        coeff = jnp.concatenate(coeff2, axis=1)
        rhat = jnp.stack(rhat3, axis=1)
        combos = jnp.stack(
            [coeff[:, ch * F:(ch + 1) * F] * rhat[:, x:x + 1]
             for x in range(3) for ch in range(2)], axis=0)
        agg1 = jax.ops.segment_sum(acat, idx_j_p, num_segments=N_T)
        cs1 = jnp.stack([jax.ops.segment_sum(combos[k], idx_j_p,
                                             num_segments=N_T)
                         for k in range(6)], axis=0)
        aggp = jnp.stack([agg1.reshape(N_T, 2, 128).transpose(1, 0, 2),
                          jnp.zeros((2, N_T, 128), f32)])
        csp = jnp.stack([cs1, jnp.zeros_like(cs1)])

    agg_specs = [
        pl.BlockSpec((1, 1, N_BLK, 128),
                     lambda i, c=c, ch=ch: (c, ch, i, 0))
        for c in range(2) for ch in range(2)
    ]
    cs_specs = [
        pl.BlockSpec((1, 1, N_BLK, 128), lambda i, c=c, k=k: (c, k, i, 0))
        for c in range(2) for k in range(6)
    ]

    grid_n = n_atoms // N_BLK
    out = pl.pallas_call(
        _node_body,
        grid=(grid_n,),
        in_specs=[_row_spec(N_BLK, F)] + agg_specs + cs_specs + [
            _row_spec(N_BLK, 128),
            _full_spec((C, F)), _full_spec((1, F)), _full_spec((F, F)),
            _full_spec((1, F)), _full_spec((F, F)), _full_spec((C, F)),
            _full_spec((F, F)), _full_spec((1, F)), _full_spec((F, F)),
            _full_spec((1, F)),
        ],
        out_specs=[_row_spec(N_BLK, F)],
        out_shape=[jax.ShapeDtypeStruct((n_atoms, F), f32)],
        interpret=_INTERPRET,
    )(q, aggp, aggp, aggp, aggp, *([csp] * 12), seg128, pw1_t, pb1, pw2_t,
      pb2, nq_t, na_t, nc_t, nb1, nw2_t, nb2)[0]
    return out


# speedup vs baseline: 7.3660x; 1.0074x over previous
import functools

import jax
import jax.numpy as jnp
from jax import lax
from jax.experimental import pallas as pl
from jax.experimental.pallas import tpu as pltpu
from jax.experimental.pallas import tpu_sc as plsc

N_ATOMS = 10000
N_PAIRS = 160000
F = 128
H = 2
C = H * F
N_RBF = 50
RBF_PAD = 64
CUTOFF = 5.0

E_BLK = 2048
N_BLK = 1000
E_P = 163840
CHUNK = 1024
N_T = 10112
ROWS_T = N_T // 16

_INTERPRET = False


def _silu(x):
    return x * jax.nn.sigmoid(x)


def _celu2(x):
    return jnp.where(x > 0, x, 2.0 * (jnp.exp(x * 0.5) - 1.0))


def _mesh():
    return plsc.VectorSubcoreMesh(core_axis_name="c", subcore_axis_name="s")


_GDN = lax.GatherDimensionNumbers(
    offset_dims=(), collapsed_slice_dims=(0,), start_index_map=(0,))


def _sc_gather(table, idx2d, B, D):
    per_w = B // 32
    iters = per_w // CHUNK

    def body(table_h, idx_h, out_h, idx_v, rows_a, rows_b, sem_ga, sem_gb,
             sem_oa, sem_ob):
        cid = lax.axis_index("c")
        sid = lax.axis_index("s")
        wid = sid * 2 + cid
        bufs = [(rows_a, sem_ga, sem_oa), (rows_b, sem_gb, sem_ob)]

        def step(i, carry):
            off = wid * per_w + i * CHUNK
            r0 = wid * (per_w // 128) + i * 8
            pltpu.sync_copy(idx_h.at[pl.ds(r0, 8)], idx_v)
            descs_g = {}
            descs_o = {}
            rv0, sg0, _ = bufs[0]
            descs_g[0] = pltpu.async_copy(table_h.at[idx_v.at[0]], rv0, sg0)
            for k in range(8):
                rv, sg, so = bufs[k % 2]
                if k + 1 < 8:
                    nrv, nsg, _ = bufs[(k + 1) % 2]
                    if k >= 1:
                        descs_o[k - 1].wait()
                    descs_g[k + 1] = pltpu.async_copy(
                        table_h.at[idx_v.at[k + 1]], nrv, nsg)
                descs_g[k].wait()
                descs_o[k] = pltpu.async_copy(
                    rv, out_h.at[pl.ds(off + k * 128, 128)], so)
            descs_o[6].wait()
            descs_o[7].wait()
            return carry

        lax.fori_loop(0, iters, step, 0)

    return pl.kernel(
        body,
        out_type=jax.ShapeDtypeStruct((B, D), jnp.float32),
        mesh=_mesh(),
        scratch_types=[
            pltpu.VMEM((8, 128), jnp.int32),
            pltpu.VMEM((128, D), jnp.float32),
            pltpu.VMEM((128, D), jnp.float32),
            pltpu.SemaphoreType.DMA,
            pltpu.SemaphoreType.DMA,
            pltpu.SemaphoreType.DMA,
            pltpu.SemaphoreType.DMA,
        ],
    )(table, idx2d)


def _sc_scatter_seg(vals, idx1d, zeros128):
    per_sc = E_P // 2
    per_t = per_sc // 16
    iters = per_t // 128

    def body(vals_h, idx_h, z_h, out_h, idx_a, idx_b, rows_a, rows_b,
             sem_a, sem_b, table):
        cid = lax.axis_index("c")
        sid = lax.axis_index("s")
        base = cid * per_sc + sid * per_t
        last = base + per_t - 128

        def fire(off, idx_v, rows_v, sem):
            o2 = jnp.minimum(off, last)
            pltpu.async_copy(vals_h.at[pl.ds(o2, 128)], rows_v, sem)
            pltpu.async_copy(idx_h.at[pl.ds(o2, 128)], idx_v, sem)

        def drain(off, idx_v, rows_v, sem):
            o2 = jnp.minimum(off, last)
            pltpu.make_async_copy(vals_h.at[pl.ds(o2, 128)], rows_v,
                                  sem).wait()
            pltpu.make_async_copy(idx_h.at[pl.ds(o2, 128)], idx_v,
                                  sem).wait()

        @pl.when(sid == 0)
        def _():
            pltpu.sync_copy(z_h, table)

        plsc.subcore_barrier()
        fire(base, idx_a, rows_a, sem_a)

        def step(i2, carry):
            off_a = base + i2 * 256
            fire(off_a + 128, idx_b, rows_b, sem_b)
            drain(off_a, idx_a, rows_a, sem_a)
            pltpu.sync_copy(rows_a, table.at[idx_a], add=True)
            fire(off_a + 256, idx_a, rows_a, sem_a)
            drain(off_a + 128, idx_b, rows_b, sem_b)
            pltpu.sync_copy(rows_b, table.at[idx_b], add=True)
            return carry

        lax.fori_loop(0, iters // 2, step, 0)
        drain(base + per_t, idx_a, rows_a, sem_a)
        plsc.subcore_barrier()

        @pl.when(sid == 0)
        def _():
            pltpu.sync_copy(table, out_h.at[cid])

    return pl.kernel(
        body,
        out_type=jax.ShapeDtypeStruct((2, N_T, 128), jnp.float32),
        mesh=_mesh(),
        scratch_types=[
            pltpu.VMEM((128,), jnp.int32),
            pltpu.VMEM((128,), jnp.int32),
            pltpu.VMEM((128, 128), jnp.float32),
            pltpu.VMEM((128, 128), jnp.float32),
            pltpu.SemaphoreType.DMA,
            pltpu.SemaphoreType.DMA,
            pltpu.VMEM_SHARED((N_T, 128), jnp.float32),
        ],
    )(vals, idx1d, zeros128)


def _sc_scatter_acat(vals, idx1d, zeros128):
    per_sc = E_P // 2
    per_t = per_sc // 16
    iters = per_t // 128

    def body(vals_h, idx_h, z_h, out_h, idx_a, idx_b, rows_a, rows_b,
             sem_a, sem_b, table):
        cid = lax.axis_index("c")
        sid = lax.axis_index("s")
        base = cid * per_sc + sid * per_t
        last = base + per_t - 128

        def fire(ch, off, idx_v, rows_v, sem):
            o2 = jnp.minimum(off, last)
            pltpu.async_copy(vals_h.at[pl.ds(o2, 128), pl.ds(ch * 128, 128)],
                             rows_v, sem)
            pltpu.async_copy(idx_h.at[pl.ds(o2, 128)], idx_v, sem)

        def drain(ch, off, idx_v, rows_v, sem):
            o2 = jnp.minimum(off, last)
            pltpu.make_async_copy(
                vals_h.at[pl.ds(o2, 128), pl.ds(ch * 128, 128)], rows_v,
                sem).wait()
            pltpu.make_async_copy(idx_h.at[pl.ds(o2, 128)], idx_v,
                                  sem).wait()

        for ch in range(2):
            @pl.when(sid == 0)
            def _():
                pltpu.sync_copy(z_h, table)

            plsc.subcore_barrier()
            fire(ch, base, idx_a, rows_a, sem_a)

            def step(i2, carry):
                off_a = base + i2 * 256
                fire(ch, off_a + 128, idx_b, rows_b, sem_b)
                drain(ch, off_a, idx_a, rows_a, sem_a)
                pltpu.sync_copy(rows_a, table.at[idx_a], add=True)
                fire(ch, off_a + 256, idx_a, rows_a, sem_a)
                drain(ch, off_a + 128, idx_b, rows_b, sem_b)
                pltpu.sync_copy(rows_b, table.at[idx_b], add=True)
                return carry

            lax.fori_loop(0, iters // 2, step, 0)
            drain(ch, base + per_t, idx_a, rows_a, sem_a)
            plsc.subcore_barrier()

            @pl.when(sid == 0)
            def _():
                pltpu.sync_copy(table, out_h.at[cid, ch])

            plsc.subcore_barrier()

    return pl.kernel(
        body,
        out_type=jax.ShapeDtypeStruct((2, 2, N_T, 128), jnp.float32),
        mesh=_mesh(),
        scratch_types=[
            pltpu.VMEM((128,), jnp.int32),
            pltpu.VMEM((128,), jnp.int32),
            pltpu.VMEM((128, 128), jnp.float32),
            pltpu.VMEM((128, 128), jnp.float32),
            pltpu.SemaphoreType.DMA,
            pltpu.SemaphoreType.DMA,
            pltpu.VMEM_SHARED((N_T, 128), jnp.float32),
        ],
    )(vals, idx1d, zeros128)


def _sc_scatter_combos(coeff2, rhat3, idx1d, zeros128):
    per_sc = E_P // 2
    per_t = per_sc // 16
    iters = per_t // 128

    def body(c0_h, c1_h, r0_h, r1_h, r2_h, idx_h, z_h, out_h, idx_a, idx_b,
             rows_a, rows_b, srows_v, rfac_a, rfac_b, sem_a, sem_b, table):
        cid = lax.axis_index("c")
        sid = lax.axis_index("s")
        coeff_hs = [c0_h, c1_h]
        rhat_hs = [r0_h, r1_h, r2_h]
        base = cid * per_sc + sid * per_t
        last = base + per_t - 128

        def fire(ch, x, off, idx_v, rows_v, rfac_v, sem):
            o2 = jnp.minimum(off, last)
            pltpu.async_copy(coeff_hs[ch].at[pl.ds(o2, 128)], rows_v, sem)
            pltpu.async_copy(idx_h.at[pl.ds(o2, 128)], idx_v, sem)
            pltpu.async_copy(rhat_hs[x].at[pl.ds(o2, 128)], rfac_v, sem)

        def drain(ch, x, off, idx_v, rows_v, rfac_v, sem):
            o2 = jnp.minimum(off, last)
            pltpu.make_async_copy(coeff_hs[ch].at[pl.ds(o2, 128)], rows_v,
                                  sem).wait()
            pltpu.make_async_copy(idx_h.at[pl.ds(o2, 128)], idx_v,
                                  sem).wait()
            pltpu.make_async_copy(rhat_hs[x].at[pl.ds(o2, 128)], rfac_v,
                                  sem).wait()

        def scale_scatter(idx_v, rows_v, rfac_v):
            def gstep(g, cc):
                base16 = (g // 2) * 16
                half = (g % 2) * 8
                rx = rfac_v[pl.ds(base16, 16)]
                for j in range(8):
                    row = base16 + half + j
                    sidx = jnp.zeros((16, 1), jnp.int32) + (half + j)
                    sc = lax.gather(
                        rx, sidx, _GDN, (1,),
                        mode=lax.GatherScatterMode.PROMISE_IN_BOUNDS)
                    for v in range(8):
                        srows_v[row, pl.ds(v * 16, 16)] = (
                            rows_v[row, pl.ds(v * 16, 16)] * sc)
                return cc

            lax.fori_loop(0, 16, gstep, 0)
            pltpu.sync_copy(srows_v, table.at[idx_v], add=True)

        for x in range(3):
            for ch in range(2):
                @pl.when(sid == 0)
                def _():
                    pltpu.sync_copy(z_h, table)

                plsc.subcore_barrier()
                fire(ch, x, base, idx_a, rows_a, rfac_a, sem_a)

                def step(i2, carry):
                    off_a = base + i2 * 256
                    fire(ch, x, off_a + 128, idx_b, rows_b, rfac_b, sem_b)
                    drain(ch, x, off_a, idx_a, rows_a, rfac_a, sem_a)
                    scale_scatter(idx_a, rows_a, rfac_a)
                    fire(ch, x, off_a + 256, idx_a, rows_a, rfac_a, sem_a)
                    drain(ch, x, off_a + 128, idx_b, rows_b, rfac_b, sem_b)
                    scale_scatter(idx_b, rows_b, rfac_b)
                    return carry

                lax.fori_loop(0, iters // 2, step, 0)
                drain(ch, x, base + per_t, idx_a, rows_a, rfac_a, sem_a)
                plsc.subcore_barrier()

                @pl.when(sid == 0)
                def _():
                    pltpu.sync_copy(table, out_h.at[cid, x * 2 + ch])

                plsc.subcore_barrier()

    return pl.kernel(
        body,
        out_type=jax.ShapeDtypeStruct((2, 6, N_T, 128), jnp.float32),
        mesh=_mesh(),
        compiler_params=pltpu.CompilerParams(needs_layout_passes=False),
        scratch_types=[
            pltpu.VMEM((128,), jnp.int32),
            pltpu.VMEM((128,), jnp.int32),
            pltpu.VMEM((128, 128), jnp.float32),
            pltpu.VMEM((128, 128), jnp.float32),
            pltpu.VMEM((128, 128), jnp.float32),
            pltpu.VMEM((128,), jnp.float32),
            pltpu.VMEM((128,), jnp.float32),
            pltpu.SemaphoreType.DMA,
            pltpu.SemaphoreType.DMA,
            pltpu.VMEM_SHARED((N_T, 128), jnp.float32),
        ],
    )(*coeff2, *rhat3, idx1d, zeros128)


def _edge1_body(n_valid, qi, qj, d, wa_t, wb_t, b_in, offs, coeff_s, w1i_t,
                w1j_t, w1f_t, w1d, b1, w2_t, b2, sem_t, semb, m_out, e_out):
    qi_v = qi[...]
    qj_v = qj[...]
    d_v = d[...]
    qe = qi_v @ wa_t[...] + qj_v @ wb_t[...] + b_in[...]
    rbf = jnp.exp(coeff_s[0, 0] * (d_v - offs[...]) ** 2)
    filt = rbf * qe
    pre = (qi_v @ w1i_t[...] + qj_v @ w1j_t[...] + filt @ w1f_t[...]
           + d_v * w1d[...] + b1[...])
    h = _silu(pre)
    m = h @ w2_t[...] + b2[...]
    att = _celu2(m @ sem_t[...] + semb[...])
    e = jnp.exp(jnp.minimum(att, 60.0))
    rows = (pl.program_id(0) * E_BLK
            + lax.broadcasted_iota(jnp.int32, (E_BLK, 1), 0))
    valid = rows < n_valid
    e128 = jnp.concatenate(
        [e, jnp.ones((E_BLK, 1), jnp.float32),
         jnp.zeros((E_BLK, 125), jnp.float32)], axis=1)
    m_out[...] = m
    e_out[...] = jnp.where(valid, e128, 0.0)


def _edge2_body(m, e128, ssg128, r, d, x0_t, x1_t, acat_out, c0_out, c1_out,
                r0_out, r1_out, r2_out):
    m_v = m[...]
    e = e128[...][:, 0:H]
    ssg = jnp.maximum(ssg128[...][:, 0:H], 1e-30)
    comb = e / ssg
    a0 = m_v * comb[:, 0:1]
    a1 = m_v * comb[:, 1:2]
    coeff = jnp.tanh(a0 @ x0_t[...] + a1 @ x1_t[...])
    acat_out[...] = jnp.concatenate([a0, a1], axis=1)
    c0_out[...] = coeff[:, 0:F]
    c1_out[...] = coeff[:, F:C]
    rhat = r[...] / (d[...] + 1e-05)
    r0_out[...] = rhat[:, 0:1]
    r1_out[...] = rhat[:, 1:2]
    r2_out[...] = rhat[:, 2:3]


def _node_body(*refs):
    (q, agg0a, agg0b, agg1a, agg1b), cs_refs = refs[0:5], refs[5:17]
    (cnt128, pw1_t, pb1, pw2_t, pb2, wq_t, wa_t, wc_t, nb1, nw2_t, nb2,
     out) = refs[17:]
    q_v = q[...]
    agg = jnp.concatenate(
        [agg0a[...][0, 0] + agg1a[...][0, 0],
         agg0b[...][0, 0] + agg1b[...][0, 0]], axis=1)
    cnt_v = jnp.maximum(cnt128[...][:, 2:3], 1.0)
    norm_parts = []
    for ch in range(2):
        acc = None
        for x in range(3):
            k = x * 2 + ch
            mean = (cs_refs[k][...][0, 0] + cs_refs[6 + k][...][0, 0]) / cnt_v
            sq = mean * mean
            acc = sq if acc is None else acc + sq
        norm_parts.append(acc)
    norm = jnp.concatenate(norm_parts, axis=1)
    qc = _silu(norm @ pw1_t[...] + pb1[...])
    qcomb = _silu(qc @ pw2_t[...] + pb2[...])
    h = _silu(q_v @ wq_t[...] + agg @ wa_t[...] + qcomb @ wc_t[...]
              + nb1[...])
    out[...] = q_v + _silu(h @ nw2_t[...] + nb2[...])


def _full_spec(shape):
    return pl.BlockSpec(shape, lambda i: (0,) * len(shape))


def _row_spec(blk, width):
    return pl.BlockSpec((blk, width), lambda i: (i, 0))


def kernel(q, mu, r_ij, d_ij, mlp_in_w, mlp_in_b, mlp_out_w1, mlp_out_b1,
           mlp_out_w2, mlp_out_b2, sem_w, sem_b, xmix_w, post_w1, post_b1,
           post_w2, post_b2, node_w1, node_b1, node_w2, node_b2, idx_i, idx_j):
    n_atoms = q.shape[0]
    n_pairs = idx_i.shape[0]
    f32 = jnp.float32

    pad_rbf = lambda a, axis: jnp.concatenate(
        [a, jnp.zeros(a.shape[:axis] + (RBF_PAD - N_RBF,) + a.shape[axis + 1:],
                      f32)], axis=axis)
    wa_t = pad_rbf(mlp_in_w[:, :F].T, 1)
    wb_t = pad_rbf(mlp_in_w[:, F:].T, 1)
    b_in = pad_rbf(mlp_in_b[None, :], 1)
    offsets = jnp.linspace(0.0, CUTOFF, N_RBF)
    offs = pad_rbf(offsets[None, :], 1)
    width = offsets[1] - offsets[0]
    coeff_s = jnp.full((1, 1), -0.5 / (width ** 2), f32)
    w1i_t = mlp_out_w1[:, :F].T
    w1j_t = mlp_out_w1[:, F:2 * F].T
    w1f_t = pad_rbf(mlp_out_w1[:, 2 * F:2 * F + N_RBF], 1).T
    w1d = mlp_out_w1[:, 2 * F + N_RBF][None, :]
    b1 = mlp_out_b1[None, :]
    w2_t = mlp_out_w2.T
    b2 = mlp_out_b2[None, :]
    sem_t = sem_w.T
    semb = sem_b[None, :]
    xm = xmix_w.reshape(C, F, H)
    x0_t = xm[:, :, 0].T
    x1_t = xm[:, :, 1].T
    pw1_t = post_w1.T
    pb1 = post_b1[None, :]
    pw2_t = post_w2.T
    pb2 = post_b2[None, :]
    nq_t = node_w1[:, :F].T
    na_perm = node_w1[:, F:F + C].reshape(F, F, H).transpose(2, 1, 0)
    na_t = na_perm.reshape(C, F)
    nc_t = node_w1[:, F + C:].T
    nb1 = node_b1[None, :]
    nw2_t = node_w2.T
    nb2 = node_b2[None, :]

    pad_e = E_P - n_pairs
    idx_i_p = jnp.concatenate([idx_i, jnp.zeros((pad_e,), jnp.int32)])
    idx_j_p = jnp.concatenate([idx_j, jnp.zeros((pad_e,), jnp.int32)])
    idxcat2d = jnp.concatenate([idx_i_p, idx_j_p]).reshape(2 * E_P // 128, 128)
    idxj2d = idx_j_p.reshape(E_P // 128, 128)
    d2 = d_ij.astype(f32)
    d_p = jnp.concatenate([d2, jnp.ones((pad_e, 1), f32)], axis=0)
    r_p = jnp.concatenate([r_ij, jnp.zeros((pad_e, 3), f32)], axis=0)
    zeros128 = jnp.zeros((N_T, 128), f32)

    grid_e = E_P // E_BLK

    if not _INTERPRET:
        gathered = _sc_gather(q, idxcat2d, 2 * E_P, F)
    else:
        gathered = jnp.concatenate([q[idx_i_p], q[idx_j_p]], axis=0)

    qi_spec = pl.BlockSpec((E_BLK, F), lambda i: (i, 0))
    qj_spec = pl.BlockSpec((E_BLK, F), lambda i: (i + grid_e, 0))
    m, e128 = pl.pallas_call(
        functools.partial(_edge1_body, n_pairs),
        grid=(grid_e,),
        in_specs=[
            qi_spec, qj_spec, _row_spec(E_BLK, 1),
            _full_spec((F, RBF_PAD)), _full_spec((F, RBF_PAD)),
            _full_spec((1, RBF_PAD)), _full_spec((1, RBF_PAD)),
            _full_spec((1, 1)),
            _full_spec((F, F)), _full_spec((F, F)), _full_spec((RBF_PAD, F)),
            _full_spec((1, F)), _full_spec((1, F)), _full_spec((F, F)),
            _full_spec((1, F)), _full_spec((F, H)), _full_spec((1, H)),
        ],
        out_specs=[_row_spec(E_BLK, F), _row_spec(E_BLK, 128)],
        out_shape=[
            jax.ShapeDtypeStruct((E_P, F), f32),
            jax.ShapeDtypeStruct((E_P, 128), f32),
        ],
        interpret=_INTERPRET,
    )(gathered, gathered, d_p, wa_t, wb_t, b_in, offs, coeff_s, w1i_t, w1j_t,
      w1f_t, w1d, b1, w2_t, b2, sem_t, semb)

    if not _INTERPRET:
        parts = _sc_scatter_seg(e128, idx_j_p, zeros128)
        seg128 = parts[0] + parts[1]
        ssg128 = _sc_gather(seg128, idxj2d, E_P, 128)
    else:
        seg128 = jax.ops.segment_sum(e128, idx_j_p, num_segments=N_T)
        ssg128 = seg128[idx_j_p]

    e2_outs = pl.pallas_call(
        _edge2_body,
        grid=(grid_e,),
        in_specs=[
            _row_spec(E_BLK, F), _row_spec(E_BLK, 128), _row_spec(E_BLK, 128),
            _row_spec(E_BLK, 3), _row_spec(E_BLK, 1),
            _full_spec((F, C)), _full_spec((F, C)),
        ],
        out_specs=([_row_spec(E_BLK, C)] + [_row_spec(E_BLK, F)] * 2
                   + [_row_spec(E_BLK, 1)] * 3),
        out_shape=([jax.ShapeDtypeStruct((E_P, C), f32)]
                   + [jax.ShapeDtypeStruct((E_P, F), f32)] * 2
                   + [jax.ShapeDtypeStruct((E_P, 1), f32)] * 3),
        interpret=_INTERPRET,
    )(m, e128, ssg128, r_p, d_p, x0_t, x1_t)
    acat = e2_outs[0]
    coeff2 = tuple(e2_outs[1:3])
    rhat3 = tuple(a.reshape(E_P) for a in e2_outs[3:6])

    if not _INTERPRET:
        aggp = _sc_scatter_acat(acat, idx_j_p, zeros128)
        csp = _sc_scatter_combos(coeff2, rhat3, idx_j_p,
                                 zeros128)
    else:
        coeff = jnp.concatenate(coeff2, axis=1)
        rhat = jnp.stack(rhat3, axis=1)
        combos = jnp.stack(
            [coeff[:, ch * F:(ch + 1) * F] * rhat[:, x:x + 1]
             for x in range(3) for ch in range(2)], axis=0)
        agg1 = jax.ops.segment_sum(acat, idx_j_p, num_segments=N_T)
        cs1 = jnp.stack([jax.ops.segment_sum(combos[k], idx_j_p,
                                             num_segments=N_T)
                         for k in range(6)], axis=0)
        aggp = jnp.stack([agg1.reshape(N_T, 2, 128).transpose(1, 0, 2),
                          jnp.zeros((2, N_T, 128), f32)])
        csp = jnp.stack([cs1, jnp.zeros_like(cs1)])

    agg_specs = [
        pl.BlockSpec((1, 1, N_BLK, 128),
                     lambda i, c=c, ch=ch: (c, ch, i, 0))
        for c in range(2) for ch in range(2)
    ]
    cs_specs = [
        pl.BlockSpec((1, 1, N_BLK, 128), lambda i, c=c, k=k: (c, k, i, 0))
        for c in range(2) for k in range(6)
    ]

    grid_n = n_atoms // N_BLK
    out = pl.pallas_call(
        _node_body,
        grid=(grid_n,),
        in_specs=[_row_spec(N_BLK, F)] + agg_specs + cs_specs + [
            _row_spec(N_BLK, 128),
            _full_spec((C, F)), _full_spec((1, F)), _full_spec((F, F)),
            _full_spec((1, F)), _full_spec((F, F)), _full_spec((C, F)),
            _full_spec((F, F)), _full_spec((1, F)), _full_spec((F, F)),
            _full_spec((1, F)),
        ],
        out_specs=[_row_spec(N_BLK, F)],
        out_shape=[jax.ShapeDtypeStruct((n_atoms, F), f32)],
        interpret=_INTERPRET,
    )(q, aggp, aggp, aggp, aggp, *([csp] * 12), seg128, pw1_t, pb1, pw2_t,
      pb2, nq_t, na_t, nc_t, nb1, nw2_t, nb2)[0]
    return out


# speedup vs baseline: 10.0854x; 1.3692x over previous
import functools

import jax
import jax.numpy as jnp
from jax import lax
from jax.experimental import pallas as pl
from jax.experimental.pallas import tpu as pltpu
from jax.experimental.pallas import tpu_sc as plsc

N_ATOMS = 10000
N_PAIRS = 160000
F = 128
H = 2
C = H * F
N_RBF = 50
RBF_PAD = 64
CUTOFF = 5.0

E_BLK = 2048
N_BLK = 1000
E_P = 163840
CHUNK = 1024
N_T = 10112
ROWS_T = N_T // 16

_INTERPRET = False


def _silu(x):
    return x * jax.nn.sigmoid(x)


def _celu2(x):
    return jnp.where(x > 0, x, 2.0 * (jnp.exp(x * 0.5) - 1.0))


def _mesh():
    return plsc.VectorSubcoreMesh(core_axis_name="c", subcore_axis_name="s")


_GDN = lax.GatherDimensionNumbers(
    offset_dims=(), collapsed_slice_dims=(0,), start_index_map=(0,))


def _sc_gather(table, idx2d, B, D):
    per_w = B // 32
    iters = per_w // CHUNK

    def body(table_h, idx_h, out_h, idx_v, rows_a, rows_b, sem_ga, sem_gb,
             sem_oa, sem_ob):
        cid = lax.axis_index("c")
        sid = lax.axis_index("s")
        wid = sid * 2 + cid
        bufs = [(rows_a, sem_ga, sem_oa), (rows_b, sem_gb, sem_ob)]

        def step(i, carry):
            off = wid * per_w + i * CHUNK
            r0 = wid * (per_w // 128) + i * 8
            pltpu.sync_copy(idx_h.at[pl.ds(r0, 8)], idx_v)
            descs_g = {}
            descs_o = {}
            rv0, sg0, _ = bufs[0]
            descs_g[0] = pltpu.async_copy(table_h.at[idx_v.at[0]], rv0, sg0)
            for k in range(8):
                rv, sg, so = bufs[k % 2]
                if k + 1 < 8:
                    nrv, nsg, _ = bufs[(k + 1) % 2]
                    if k >= 1:
                        descs_o[k - 1].wait()
                    descs_g[k + 1] = pltpu.async_copy(
                        table_h.at[idx_v.at[k + 1]], nrv, nsg)
                descs_g[k].wait()
                descs_o[k] = pltpu.async_copy(
                    rv, out_h.at[pl.ds(off + k * 128, 128)], so)
            descs_o[6].wait()
            descs_o[7].wait()
            return carry

        lax.fori_loop(0, iters, step, 0)

    return pl.kernel(
        body,
        out_type=jax.ShapeDtypeStruct((B, D), jnp.float32),
        mesh=_mesh(),
        scratch_types=[
            pltpu.VMEM((8, 128), jnp.int32),
            pltpu.VMEM((128, D), jnp.float32),
            pltpu.VMEM((128, D), jnp.float32),
            pltpu.SemaphoreType.DMA,
            pltpu.SemaphoreType.DMA,
            pltpu.SemaphoreType.DMA,
            pltpu.SemaphoreType.DMA,
        ],
    )(table, idx2d)


def _sc_scatter_seg(vals, idx1d, zeros128):
    per_sc = E_P // 2
    per_t = per_sc // 16
    iters = per_t // 128

    def body(vals_h, idx_h, z_h, out_h, idx_a, idx_b, rows_a, rows_b,
             sem_a, sem_b, table):
        cid = lax.axis_index("c")
        sid = lax.axis_index("s")
        base = cid * per_sc + sid * per_t
        last = base + per_t - 128

        def fire(off, idx_v, rows_v, sem):
            o2 = jnp.minimum(off, last)
            pltpu.async_copy(vals_h.at[pl.ds(o2, 128)], rows_v, sem)
            pltpu.async_copy(idx_h.at[pl.ds(o2, 128)], idx_v, sem)

        def drain(off, idx_v, rows_v, sem):
            o2 = jnp.minimum(off, last)
            pltpu.make_async_copy(vals_h.at[pl.ds(o2, 128)], rows_v,
                                  sem).wait()
            pltpu.make_async_copy(idx_h.at[pl.ds(o2, 128)], idx_v,
                                  sem).wait()

        @pl.when(sid == 0)
        def _():
            pltpu.sync_copy(z_h, table)

        plsc.subcore_barrier()
        fire(base, idx_a, rows_a, sem_a)

        def step(i2, carry):
            off_a = base + i2 * 256
            fire(off_a + 128, idx_b, rows_b, sem_b)
            drain(off_a, idx_a, rows_a, sem_a)
            pltpu.sync_copy(rows_a, table.at[idx_a], add=True)
            fire(off_a + 256, idx_a, rows_a, sem_a)
            drain(off_a + 128, idx_b, rows_b, sem_b)
            pltpu.sync_copy(rows_b, table.at[idx_b], add=True)
            return carry

        lax.fori_loop(0, iters // 2, step, 0)
        drain(base + per_t, idx_a, rows_a, sem_a)
        plsc.subcore_barrier()

        @pl.when(sid == 0)
        def _():
            pltpu.sync_copy(table, out_h.at[cid])

    return pl.kernel(
        body,
        out_type=jax.ShapeDtypeStruct((2, N_T, 128), jnp.float32),
        mesh=_mesh(),
        scratch_types=[
            pltpu.VMEM((128,), jnp.int32),
            pltpu.VMEM((128,), jnp.int32),
            pltpu.VMEM((128, 128), jnp.float32),
            pltpu.VMEM((128, 128), jnp.float32),
            pltpu.SemaphoreType.DMA,
            pltpu.SemaphoreType.DMA,
            pltpu.VMEM_SHARED((N_T, 128), jnp.float32),
        ],
    )(vals, idx1d, zeros128)


def _sc_scatter_acat(vals, idx1d, zeros128):
    per_sc = E_P // 2
    per_t = per_sc // 16
    iters = per_t // 128

    def body(vals_h, idx_h, z_h, out_h, idx_a, idx_b, rows_a, rows_b,
             sem_a, sem_b, table):
        cid = lax.axis_index("c")
        sid = lax.axis_index("s")
        base = cid * per_sc + sid * per_t
        last = base + per_t - 128

        def fire(ch, off, idx_v, rows_v, sem):
            o2 = jnp.minimum(off, last)
            pltpu.async_copy(vals_h.at[pl.ds(o2, 128), pl.ds(ch * 128, 128)],
                             rows_v, sem)
            pltpu.async_copy(idx_h.at[pl.ds(o2, 128)], idx_v, sem)

        def drain(ch, off, idx_v, rows_v, sem):
            o2 = jnp.minimum(off, last)
            pltpu.make_async_copy(
                vals_h.at[pl.ds(o2, 128), pl.ds(ch * 128, 128)], rows_v,
                sem).wait()
            pltpu.make_async_copy(idx_h.at[pl.ds(o2, 128)], idx_v,
                                  sem).wait()

        for ch in range(2):
            @pl.when(sid == 0)
            def _():
                pltpu.sync_copy(z_h, table)

            plsc.subcore_barrier()
            fire(ch, base, idx_a, rows_a, sem_a)

            def step(i2, carry):
                off_a = base + i2 * 256
                fire(ch, off_a + 128, idx_b, rows_b, sem_b)
                drain(ch, off_a, idx_a, rows_a, sem_a)
                pltpu.sync_copy(rows_a, table.at[idx_a], add=True)
                fire(ch, off_a + 256, idx_a, rows_a, sem_a)
                drain(ch, off_a + 128, idx_b, rows_b, sem_b)
                pltpu.sync_copy(rows_b, table.at[idx_b], add=True)
                return carry

            lax.fori_loop(0, iters // 2, step, 0)
            drain(ch, base + per_t, idx_a, rows_a, sem_a)
            plsc.subcore_barrier()

            @pl.when(sid == 0)
            def _():
                pltpu.sync_copy(table, out_h.at[cid, ch])

            plsc.subcore_barrier()

    return pl.kernel(
        body,
        out_type=jax.ShapeDtypeStruct((2, 2, N_T, 128), jnp.float32),
        mesh=_mesh(),
        scratch_types=[
            pltpu.VMEM((128,), jnp.int32),
            pltpu.VMEM((128,), jnp.int32),
            pltpu.VMEM((128, 128), jnp.float32),
            pltpu.VMEM((128, 128), jnp.float32),
            pltpu.SemaphoreType.DMA,
            pltpu.SemaphoreType.DMA,
            pltpu.VMEM_SHARED((N_T, 128), jnp.float32),
        ],
    )(vals, idx1d, zeros128)


def _sc_scatter_combos(coeff2, rhat3, idx1d, zeros128):
    per_sc = E_P // 2
    per_t = per_sc // 16
    iters = per_t // 128

    def body(c0_h, c1_h, r0_h, r1_h, r2_h, idx_h, z_h, out_h, idx_a, idx_b,
             rows_a, rows_b, rfac_a, rfac_b, sem_a, sem_b, sem_sa, sem_sb,
             table):
        cid = lax.axis_index("c")
        sid = lax.axis_index("s")
        coeff_hs = [c0_h, c1_h]
        rhat_hs = [r0_h, r1_h, r2_h]
        base = cid * per_sc + sid * per_t
        last = base + per_t - 128

        def fire(ch, x, off, idx_v, rows_v, rfac_v, sem):
            o2 = jnp.minimum(off, last)
            pltpu.async_copy(coeff_hs[ch].at[pl.ds(o2, 128)], rows_v, sem)
            pltpu.async_copy(idx_h.at[pl.ds(o2, 128)], idx_v, sem)
            pltpu.async_copy(rhat_hs[x].at[pl.ds(o2, 128)], rfac_v, sem)

        def drain(ch, x, off, idx_v, rows_v, rfac_v, sem):
            o2 = jnp.minimum(off, last)
            pltpu.make_async_copy(coeff_hs[ch].at[pl.ds(o2, 128)], rows_v,
                                  sem).wait()
            pltpu.make_async_copy(idx_h.at[pl.ds(o2, 128)], idx_v,
                                  sem).wait()
            pltpu.make_async_copy(rhat_hs[x].at[pl.ds(o2, 128)], rfac_v,
                                  sem).wait()

        def scale(idx_v, rows_v, rfac_v):
            def gstep(g, cc):
                base16 = (g // 2) * 16
                half = (g % 2) * 8
                rx = rfac_v[pl.ds(base16, 16)]
                for j in range(8):
                    row = base16 + half + j
                    sidx = jnp.zeros((16, 1), jnp.int32) + (half + j)
                    sc = lax.gather(
                        rx, sidx, _GDN, (1,),
                        mode=lax.GatherScatterMode.PROMISE_IN_BOUNDS)
                    for v in range(8):
                        rows_v[row, pl.ds(v * 16, 16)] = (
                            rows_v[row, pl.ds(v * 16, 16)] * sc)
                return cc

            lax.fori_loop(0, 16, gstep, 0)

        def drain_scatter(idx_v, rows_v, sem):
            pltpu.make_async_copy(rows_v, table.at[idx_v], sem).wait()

        for x in range(3):
            for ch in range(2):
                @pl.when(sid == 0)
                def _():
                    pltpu.sync_copy(z_h, table)

                plsc.subcore_barrier()
                fire(ch, x, base, idx_a, rows_a, rfac_a, sem_a)

                def step(i2, carry):
                    off_a = base + i2 * 256

                    @pl.when(i2 > 0)
                    def _():
                        drain_scatter(idx_b, rows_b, sem_sb)

                    fire(ch, x, off_a + 128, idx_b, rows_b, rfac_b, sem_b)
                    drain(ch, x, off_a, idx_a, rows_a, rfac_a, sem_a)
                    scale(idx_a, rows_a, rfac_a)
                    pltpu.async_copy(rows_a, table.at[idx_a], sem_sa,
                                     add=True)
                    drain(ch, x, off_a + 128, idx_b, rows_b, rfac_b, sem_b)
                    scale(idx_b, rows_b, rfac_b)
                    drain_scatter(idx_a, rows_a, sem_sa)
                    fire(ch, x, off_a + 256, idx_a, rows_a, rfac_a, sem_a)
                    pltpu.async_copy(rows_b, table.at[idx_b], sem_sb,
                                     add=True)
                    return carry

                lax.fori_loop(0, iters // 2, step, 0)
                drain(ch, x, base + per_t, idx_a, rows_a, rfac_a, sem_a)
                drain_scatter(idx_b, rows_b, sem_sb)
                plsc.subcore_barrier()

                @pl.when(sid == 0)
                def _():
                    pltpu.sync_copy(table, out_h.at[cid, x * 2 + ch])

                plsc.subcore_barrier()

    return pl.kernel(
        body,
        out_type=jax.ShapeDtypeStruct((2, 6, N_T, 128), jnp.float32),
        mesh=_mesh(),
        compiler_params=pltpu.CompilerParams(needs_layout_passes=False),
        scratch_types=[
            pltpu.VMEM((128,), jnp.int32),
            pltpu.VMEM((128,), jnp.int32),
            pltpu.VMEM((128, 128), jnp.float32),
            pltpu.VMEM((128, 128), jnp.float32),
            pltpu.VMEM((128,), jnp.float32),
            pltpu.VMEM((128,), jnp.float32),
            pltpu.SemaphoreType.DMA,
            pltpu.SemaphoreType.DMA,
            pltpu.SemaphoreType.DMA,
            pltpu.SemaphoreType.DMA,
            pltpu.VMEM_SHARED((N_T, 128), jnp.float32),
        ],
    )(*coeff2, *rhat3, idx1d, zeros128)


def _edge1_body(n_valid, qi, qj, d, wa_t, wb_t, b_in, offs, coeff_s, w1i_t,
                w1j_t, w1f_t, w1d, b1, w2_t, b2, sem_t, semb, m_out, e_out):
    qi_v = qi[...]
    qj_v = qj[...]
    d_v = d[...]
    qe = qi_v @ wa_t[...] + qj_v @ wb_t[...] + b_in[...]
    rbf = jnp.exp(coeff_s[0, 0] * (d_v - offs[...]) ** 2)
    filt = rbf * qe
    pre = (qi_v @ w1i_t[...] + qj_v @ w1j_t[...] + filt @ w1f_t[...]
           + d_v * w1d[...] + b1[...])
    h = _silu(pre)
    m = h @ w2_t[...] + b2[...]
    att = _celu2(m @ sem_t[...] + semb[...])
    e = jnp.exp(jnp.minimum(att, 60.0))
    rows = (pl.program_id(0) * E_BLK
            + lax.broadcasted_iota(jnp.int32, (E_BLK, 1), 0))
    valid = rows < n_valid
    e128 = jnp.concatenate(
        [e, jnp.ones((E_BLK, 1), jnp.float32),
         jnp.zeros((E_BLK, 125), jnp.float32)], axis=1)
    m_out[...] = m
    e_out[...] = jnp.where(valid, e128, 0.0)


def _edge2_body(m, e128, ssg128, r, d, x0_t, x1_t, acat_out, c0_out, c1_out,
                r0_out, r1_out, r2_out):
    m_v = m[...]
    e = e128[...][:, 0:H]
    ssg = jnp.maximum(ssg128[...][:, 0:H], 1e-30)
    comb = e / ssg
    a0 = m_v * comb[:, 0:1]
    a1 = m_v * comb[:, 1:2]
    coeff = jnp.tanh(a0 @ x0_t[...] + a1 @ x1_t[...])
    acat_out[...] = jnp.concatenate([a0, a1], axis=1)
    c0_out[...] = coeff[:, 0:F]
    c1_out[...] = coeff[:, F:C]
    rhat = r[...] / (d[...] + 1e-05)
    r0_out[...] = rhat[:, 0:1]
    r1_out[...] = rhat[:, 1:2]
    r2_out[...] = rhat[:, 2:3]


def _node_body(*refs):
    (q, agg0a, agg0b, agg1a, agg1b), cs_refs = refs[0:5], refs[5:17]
    (cnt128, pw1_t, pb1, pw2_t, pb2, wq_t, wa_t, wc_t, nb1, nw2_t, nb2,
     out) = refs[17:]
    q_v = q[...]
    agg = jnp.concatenate(
        [agg0a[...][0, 0] + agg1a[...][0, 0],
         agg0b[...][0, 0] + agg1b[...][0, 0]], axis=1)
    cnt_v = jnp.maximum(cnt128[...][:, 2:3], 1.0)
    norm_parts = []
    for ch in range(2):
        acc = None
        for x in range(3):
            k = x * 2 + ch
            mean = (cs_refs[k][...][0, 0] + cs_refs[6 + k][...][0, 0]) / cnt_v
            sq = mean * mean
            acc = sq if acc is None else acc + sq
        norm_parts.append(acc)
    norm = jnp.concatenate(norm_parts, axis=1)
    qc = _silu(norm @ pw1_t[...] + pb1[...])
    qcomb = _silu(qc @ pw2_t[...] + pb2[...])
    h = _silu(q_v @ wq_t[...] + agg @ wa_t[...] + qcomb @ wc_t[...]
              + nb1[...])
    out[...] = q_v + _silu(h @ nw2_t[...] + nb2[...])


def _full_spec(shape):
    return pl.BlockSpec(shape, lambda i: (0,) * len(shape))


def _row_spec(blk, width):
    return pl.BlockSpec((blk, width), lambda i: (i, 0))


def kernel(q, mu, r_ij, d_ij, mlp_in_w, mlp_in_b, mlp_out_w1, mlp_out_b1,
           mlp_out_w2, mlp_out_b2, sem_w, sem_b, xmix_w, post_w1, post_b1,
           post_w2, post_b2, node_w1, node_b1, node_w2, node_b2, idx_i, idx_j):
    n_atoms = q.shape[0]
    n_pairs = idx_i.shape[0]
    f32 = jnp.float32

    pad_rbf = lambda a, axis: jnp.concatenate(
        [a, jnp.zeros(a.shape[:axis] + (RBF_PAD - N_RBF,) + a.shape[axis + 1:],
                      f32)], axis=axis)
    wa_t = pad_rbf(mlp_in_w[:, :F].T, 1)
    wb_t = pad_rbf(mlp_in_w[:, F:].T, 1)
    b_in = pad_rbf(mlp_in_b[None, :], 1)
    offsets = jnp.linspace(0.0, CUTOFF, N_RBF)
    offs = pad_rbf(offsets[None, :], 1)
    width = offsets[1] - offsets[0]
    coeff_s = jnp.full((1, 1), -0.5 / (width ** 2), f32)
    w1i_t = mlp_out_w1[:, :F].T
    w1j_t = mlp_out_w1[:, F:2 * F].T
    w1f_t = pad_rbf(mlp_out_w1[:, 2 * F:2 * F + N_RBF], 1).T
    w1d = mlp_out_w1[:, 2 * F + N_RBF][None, :]
    b1 = mlp_out_b1[None, :]
    w2_t = mlp_out_w2.T
    b2 = mlp_out_b2[None, :]
    sem_t = sem_w.T
    semb = sem_b[None, :]
    xm = xmix_w.reshape(C, F, H)
    x0_t = xm[:, :, 0].T
    x1_t = xm[:, :, 1].T
    pw1_t = post_w1.T
    pb1 = post_b1[None, :]
    pw2_t = post_w2.T
    pb2 = post_b2[None, :]
    nq_t = node_w1[:, :F].T
    na_perm = node_w1[:, F:F + C].reshape(F, F, H).transpose(2, 1, 0)
    na_t = na_perm.reshape(C, F)
    nc_t = node_w1[:, F + C:].T
    nb1 = node_b1[None, :]
    nw2_t = node_w2.T
    nb2 = node_b2[None, :]

    pad_e = E_P - n_pairs
    idx_i_p = jnp.concatenate([idx_i, jnp.zeros((pad_e,), jnp.int32)])
    idx_j_p = jnp.concatenate([idx_j, jnp.zeros((pad_e,), jnp.int32)])
    idxcat2d = jnp.concatenate([idx_i_p, idx_j_p]).reshape(2 * E_P // 128, 128)
    idxj2d = idx_j_p.reshape(E_P // 128, 128)
    d2 = d_ij.astype(f32)
    d_p = jnp.concatenate([d2, jnp.ones((pad_e, 1), f32)], axis=0)
    r_p = jnp.concatenate([r_ij, jnp.zeros((pad_e, 3), f32)], axis=0)
    zeros128 = jnp.zeros((N_T, 128), f32)

    grid_e = E_P // E_BLK

    if not _INTERPRET:
        gathered = _sc_gather(q, idxcat2d, 2 * E_P, F)
    else:
        gathered = jnp.concatenate([q[idx_i_p], q[idx_j_p]], axis=0)

    qi_spec = pl.BlockSpec((E_BLK, F), lambda i: (i, 0))
    qj_spec = pl.BlockSpec((E_BLK, F), lambda i: (i + grid_e, 0))
    m, e128 = pl.pallas_call(
        functools.partial(_edge1_body, n_pairs),
        grid=(grid_e,),
        in_specs=[
            qi_spec, qj_spec, _row_spec(E_BLK, 1),
            _full_spec((F, RBF_PAD)), _full_spec((F, RBF_PAD)),
            _full_spec((1, RBF_PAD)), _full_spec((1, RBF_PAD)),
            _full_spec((1, 1)),
            _full_spec((F, F)), _full_spec((F, F)), _full_spec((RBF_PAD, F)),
            _full_spec((1, F)), _full_spec((1, F)), _full_spec((F, F)),
            _full_spec((1, F)), _full_spec((F, H)), _full_spec((1, H)),
        ],
        out_specs=[_row_spec(E_BLK, F), _row_spec(E_BLK, 128)],
        out_shape=[
            jax.ShapeDtypeStruct((E_P, F), f32),
            jax.ShapeDtypeStruct((E_P, 128), f32),
        ],
        interpret=_INTERPRET,
    )(gathered, gathered, d_p, wa_t, wb_t, b_in, offs, coeff_s, w1i_t, w1j_t,
      w1f_t, w1d, b1, w2_t, b2, sem_t, semb)

    if not _INTERPRET:
        parts = _sc_scatter_seg(e128, idx_j_p, zeros128)
        seg128 = parts[0] + parts[1]
        ssg128 = _sc_gather(seg128, idxj2d, E_P, 128)
    else:
        seg128 = jax.ops.segment_sum(e128, idx_j_p, num_segments=N_T)
        ssg128 = seg128[idx_j_p]

    e2_outs = pl.pallas_call(
        _edge2_body,
        grid=(grid_e,),
        in_specs=[
            _row_spec(E_BLK, F), _row_spec(E_BLK, 128), _row_spec(E_BLK, 128),
            _row_spec(E_BLK, 3), _row_spec(E_BLK, 1),
            _full_spec((F, C)), _full_spec((F, C)),
        ],
        out_specs=([_row_spec(E_BLK, C)] + [_row_spec(E_BLK, F)] * 2
                   + [_row_spec(E_BLK, 1)] * 3),
        out_shape=([jax.ShapeDtypeStruct((E_P, C), f32)]
                   + [jax.ShapeDtypeStruct((E_P, F), f32)] * 2
                   + [jax.ShapeDtypeStruct((E_P, 1), f32)] * 3),
        interpret=_INTERPRET,
    )(m, e128, ssg128, r_p, d_p, x0_t, x1_t)
    acat = e2_outs[0]
    coeff2 = tuple(e2_outs[1:3])
    rhat3 = tuple(a.reshape(E_P) for a in e2_outs[3:6])

    if not _INTERPRET:
        aggp = _sc_scatter_acat(acat, idx_j_p, zeros128)
        csp = _sc_scatter_combos(coeff2, rhat3, idx_j_p,
                                 zeros128)
    else:
        coeff = jnp.concatenate(coeff2, axis=1)
        rhat = jnp.stack(rhat3, axis=1)
        combos = jnp.stack(
            [coeff[:, ch * F:(ch + 1) * F] * rhat[:, x:x + 1]
             for x in range(3) for ch in range(2)], axis=0)
        agg1 = jax.ops.segment_sum(acat, idx_j_p, num_segments=N_T)
        cs1 = jnp.stack([jax.ops.segment_sum(combos[k], idx_j_p,
                                             num_segments=N_T)
                         for k in range(6)], axis=0)
        aggp = jnp.stack([agg1.reshape(N_T, 2, 128).transpose(1, 0, 2),
                          jnp.zeros((2, N_T, 128), f32)])
        csp = jnp.stack([cs1, jnp.zeros_like(cs1)])

    agg_specs = [
        pl.BlockSpec((1, 1, N_BLK, 128),
                     lambda i, c=c, ch=ch: (c, ch, i, 0))
        for c in range(2) for ch in range(2)
    ]
    cs_specs = [
        pl.BlockSpec((1, 1, N_BLK, 128), lambda i, c=c, k=k: (c, k, i, 0))
        for c in range(2) for k in range(6)
    ]

    grid_n = n_atoms // N_BLK
    out = pl.pallas_call(
        _node_body,
        grid=(grid_n,),
        in_specs=[_row_spec(N_BLK, F)] + agg_specs + cs_specs + [
            _row_spec(N_BLK, 128),
            _full_spec((C, F)), _full_spec((1, F)), _full_spec((F, F)),
            _full_spec((1, F)), _full_spec((F, F)), _full_spec((C, F)),
            _full_spec((F, F)), _full_spec((1, F)), _full_spec((F, F)),
            _full_spec((1, F)),
        ],
        out_specs=[_row_spec(N_BLK, F)],
        out_shape=[jax.ShapeDtypeStruct((n_atoms, F), f32)],
        interpret=_INTERPRET,
    )(q, aggp, aggp, aggp, aggp, *([csp] * 12), seg128, pw1_t, pb1, pw2_t,
      pb2, nq_t, na_t, nc_t, nb1, nw2_t, nb2)[0]
    return out


# speedup vs baseline: 10.0957x; 1.0010x over previous
import functools

import jax
import jax.numpy as jnp
from jax import lax
from jax.experimental import pallas as pl
from jax.experimental.pallas import tpu as pltpu
from jax.experimental.pallas import tpu_sc as plsc

N_ATOMS = 10000
N_PAIRS = 160000
F = 128
H = 2
C = H * F
N_RBF = 50
RBF_PAD = 64
CUTOFF = 5.0

E_BLK = 2048
N_BLK = 1000
E_P = 163840
CHUNK = 1024
N_T = 10112
ROWS_T = N_T // 16

def _silu(x):
    return x * jax.nn.sigmoid(x)


def _celu2(x):
    return jnp.where(x > 0, x, 2.0 * (jnp.exp(x * 0.5) - 1.0))


def _mesh():
    return plsc.VectorSubcoreMesh(core_axis_name="c", subcore_axis_name="s")


_GDN = lax.GatherDimensionNumbers(
    offset_dims=(), collapsed_slice_dims=(0,), start_index_map=(0,))


def _sc_gather(table, idx2d, B, D):
    per_w = B // 32
    iters = per_w // CHUNK

    def body(table_h, idx_h, out_h, idx_v, rows_a, rows_b, sem_ga, sem_gb,
             sem_oa, sem_ob):
        cid = lax.axis_index("c")
        sid = lax.axis_index("s")
        wid = sid * 2 + cid
        bufs = [(rows_a, sem_ga, sem_oa), (rows_b, sem_gb, sem_ob)]

        def step(i, carry):
            off = wid * per_w + i * CHUNK
            r0 = wid * (per_w // 128) + i * 8
            pltpu.sync_copy(idx_h.at[pl.ds(r0, 8)], idx_v)
            descs_g = {}
            descs_o = {}
            rv0, sg0, _ = bufs[0]
            descs_g[0] = pltpu.async_copy(table_h.at[idx_v.at[0]], rv0, sg0)
            for k in range(8):
                rv, sg, so = bufs[k % 2]
                if k + 1 < 8:
                    nrv, nsg, _ = bufs[(k + 1) % 2]
                    if k >= 1:
                        descs_o[k - 1].wait()
                    descs_g[k + 1] = pltpu.async_copy(
                        table_h.at[idx_v.at[k + 1]], nrv, nsg)
                descs_g[k].wait()
                descs_o[k] = pltpu.async_copy(
                    rv, out_h.at[pl.ds(off + k * 128, 128)], so)
            descs_o[6].wait()
            descs_o[7].wait()
            return carry

        lax.fori_loop(0, iters, step, 0)

    return pl.kernel(
        body,
        out_type=jax.ShapeDtypeStruct((B, D), jnp.float32),
        mesh=_mesh(),
        scratch_types=[
            pltpu.VMEM((8, 128), jnp.int32),
            pltpu.VMEM((128, D), jnp.float32),
            pltpu.VMEM((128, D), jnp.float32),
            pltpu.SemaphoreType.DMA,
            pltpu.SemaphoreType.DMA,
            pltpu.SemaphoreType.DMA,
            pltpu.SemaphoreType.DMA,
        ],
    )(table, idx2d)


def _sc_scatter_seg(vals, idx1d, zeros128):
    per_sc = E_P // 2
    per_t = per_sc // 16
    iters = per_t // 128

    def body(vals_h, idx_h, z_h, out_h, idx_a, idx_b, rows_a, rows_b,
             sem_a, sem_b, table):
        cid = lax.axis_index("c")
        sid = lax.axis_index("s")
        base = cid * per_sc + sid * per_t
        last = base + per_t - 128

        def fire(off, idx_v, rows_v, sem):
            o2 = jnp.minimum(off, last)
            pltpu.async_copy(vals_h.at[pl.ds(o2, 128)], rows_v, sem)
            pltpu.async_copy(idx_h.at[pl.ds(o2, 128)], idx_v, sem)

        def drain(off, idx_v, rows_v, sem):
            o2 = jnp.minimum(off, last)
            pltpu.make_async_copy(vals_h.at[pl.ds(o2, 128)], rows_v,
                                  sem).wait()
            pltpu.make_async_copy(idx_h.at[pl.ds(o2, 128)], idx_v,
                                  sem).wait()

        @pl.when(sid == 0)
        def _():
            pltpu.sync_copy(z_h, table)

        plsc.subcore_barrier()
        fire(base, idx_a, rows_a, sem_a)

        def step(i2, carry):
            off_a = base + i2 * 256
            fire(off_a + 128, idx_b, rows_b, sem_b)
            drain(off_a, idx_a, rows_a, sem_a)
            pltpu.sync_copy(rows_a, table.at[idx_a], add=True)
            fire(off_a + 256, idx_a, rows_a, sem_a)
            drain(off_a + 128, idx_b, rows_b, sem_b)
            pltpu.sync_copy(rows_b, table.at[idx_b], add=True)
            return carry

        lax.fori_loop(0, iters // 2, step, 0)
        drain(base + per_t, idx_a, rows_a, sem_a)
        plsc.subcore_barrier()

        @pl.when(sid == 0)
        def _():
            pltpu.sync_copy(table, out_h.at[cid])

    return pl.kernel(
        body,
        out_type=jax.ShapeDtypeStruct((2, N_T, 128), jnp.float32),
        mesh=_mesh(),
        scratch_types=[
            pltpu.VMEM((128,), jnp.int32),
            pltpu.VMEM((128,), jnp.int32),
            pltpu.VMEM((128, 128), jnp.float32),
            pltpu.VMEM((128, 128), jnp.float32),
            pltpu.SemaphoreType.DMA,
            pltpu.SemaphoreType.DMA,
            pltpu.VMEM_SHARED((N_T, 128), jnp.float32),
        ],
    )(vals, idx1d, zeros128)


def _sc_scatter_acat(vals, idx1d, zeros128):
    per_sc = E_P // 2
    per_t = per_sc // 16
    iters = per_t // 128

    def body(vals_h, idx_h, z_h, out_h, idx_a, idx_b, rows_a, rows_b,
             sem_a, sem_b, table):
        cid = lax.axis_index("c")
        sid = lax.axis_index("s")
        base = cid * per_sc + sid * per_t
        last = base + per_t - 128

        def fire(ch, off, idx_v, rows_v, sem):
            o2 = jnp.minimum(off, last)
            pltpu.async_copy(vals_h.at[pl.ds(o2, 128), pl.ds(ch * 128, 128)],
                             rows_v, sem)
            pltpu.async_copy(idx_h.at[pl.ds(o2, 128)], idx_v, sem)

        def drain(ch, off, idx_v, rows_v, sem):
            o2 = jnp.minimum(off, last)
            pltpu.make_async_copy(
                vals_h.at[pl.ds(o2, 128), pl.ds(ch * 128, 128)], rows_v,
                sem).wait()
            pltpu.make_async_copy(idx_h.at[pl.ds(o2, 128)], idx_v,
                                  sem).wait()

        for ch in range(2):
            @pl.when(sid == 0)
            def _():
                pltpu.sync_copy(z_h, table)

            plsc.subcore_barrier()
            fire(ch, base, idx_a, rows_a, sem_a)

            def step(i2, carry):
                off_a = base + i2 * 256
                fire(ch, off_a + 128, idx_b, rows_b, sem_b)
                drain(ch, off_a, idx_a, rows_a, sem_a)
                pltpu.sync_copy(rows_a, table.at[idx_a], add=True)
                fire(ch, off_a + 256, idx_a, rows_a, sem_a)
                drain(ch, off_a + 128, idx_b, rows_b, sem_b)
                pltpu.sync_copy(rows_b, table.at[idx_b], add=True)
                return carry

            lax.fori_loop(0, iters // 2, step, 0)
            drain(ch, base + per_t, idx_a, rows_a, sem_a)
            plsc.subcore_barrier()

            @pl.when(sid == 0)
            def _():
                pltpu.sync_copy(table, out_h.at[cid, ch])

            plsc.subcore_barrier()

    return pl.kernel(
        body,
        out_type=jax.ShapeDtypeStruct((2, 2, N_T, 128), jnp.float32),
        mesh=_mesh(),
        scratch_types=[
            pltpu.VMEM((128,), jnp.int32),
            pltpu.VMEM((128,), jnp.int32),
            pltpu.VMEM((128, 128), jnp.float32),
            pltpu.VMEM((128, 128), jnp.float32),
            pltpu.SemaphoreType.DMA,
            pltpu.SemaphoreType.DMA,
            pltpu.VMEM_SHARED((N_T, 128), jnp.float32),
        ],
    )(vals, idx1d, zeros128)


def _sc_scatter_combos(coeff2, rhat3, idx1d, zeros128):
    per_sc = E_P // 2
    per_t = per_sc // 16
    iters = per_t // 128

    def body(c0_h, c1_h, r0_h, r1_h, r2_h, idx_h, z_h, out_h, idx_a, idx_b,
             rows_a, rows_b, rfac_a, rfac_b, sem_a, sem_b, sem_sa, sem_sb,
             table):
        cid = lax.axis_index("c")
        sid = lax.axis_index("s")
        coeff_hs = [c0_h, c1_h]
        rhat_hs = [r0_h, r1_h, r2_h]
        base = cid * per_sc + sid * per_t
        last = base + per_t - 128

        def fire(ch, x, off, idx_v, rows_v, rfac_v, sem):
            o2 = jnp.minimum(off, last)
            pltpu.async_copy(coeff_hs[ch].at[pl.ds(o2, 128)], rows_v, sem)
            pltpu.async_copy(idx_h.at[pl.ds(o2, 128)], idx_v, sem)
            pltpu.async_copy(rhat_hs[x].at[pl.ds(o2, 128)], rfac_v, sem)

        def drain(ch, x, off, idx_v, rows_v, rfac_v, sem):
            o2 = jnp.minimum(off, last)
            pltpu.make_async_copy(coeff_hs[ch].at[pl.ds(o2, 128)], rows_v,
                                  sem).wait()
            pltpu.make_async_copy(idx_h.at[pl.ds(o2, 128)], idx_v,
                                  sem).wait()
            pltpu.make_async_copy(rhat_hs[x].at[pl.ds(o2, 128)], rfac_v,
                                  sem).wait()

        def scale(idx_v, rows_v, rfac_v):
            def gstep(g, cc):
                base16 = (g // 2) * 16
                half = (g % 2) * 8
                rx = rfac_v[pl.ds(base16, 16)]
                for j in range(8):
                    row = base16 + half + j
                    sidx = jnp.zeros((16, 1), jnp.int32) + (half + j)
                    sc = lax.gather(
                        rx, sidx, _GDN, (1,),
                        mode=lax.GatherScatterMode.PROMISE_IN_BOUNDS)
                    for v in range(8):
                        rows_v[row, pl.ds(v * 16, 16)] = (
                            rows_v[row, pl.ds(v * 16, 16)] * sc)
                return cc

            lax.fori_loop(0, 16, gstep, 0)

        def drain_scatter(idx_v, rows_v, sem):
            pltpu.make_async_copy(rows_v, table.at[idx_v], sem).wait()

        for x in range(3):
            for ch in range(2):
                @pl.when(sid == 0)
                def _():
                    pltpu.sync_copy(z_h, table)

                plsc.subcore_barrier()
                fire(ch, x, base, idx_a, rows_a, rfac_a, sem_a)

                def step(i2, carry):
                    off_a = base + i2 * 256

                    @pl.when(i2 > 0)
                    def _():
                        drain_scatter(idx_b, rows_b, sem_sb)

                    fire(ch, x, off_a + 128, idx_b, rows_b, rfac_b, sem_b)
                    drain(ch, x, off_a, idx_a, rows_a, rfac_a, sem_a)
                    scale(idx_a, rows_a, rfac_a)
                    pltpu.async_copy(rows_a, table.at[idx_a], sem_sa,
                                     add=True)
                    drain(ch, x, off_a + 128, idx_b, rows_b, rfac_b, sem_b)
                    scale(idx_b, rows_b, rfac_b)
                    drain_scatter(idx_a, rows_a, sem_sa)
                    fire(ch, x, off_a + 256, idx_a, rows_a, rfac_a, sem_a)
                    pltpu.async_copy(rows_b, table.at[idx_b], sem_sb,
                                     add=True)
                    return carry

                lax.fori_loop(0, iters // 2, step, 0)
                drain(ch, x, base + per_t, idx_a, rows_a, rfac_a, sem_a)
                drain_scatter(idx_b, rows_b, sem_sb)
                plsc.subcore_barrier()

                @pl.when(sid == 0)
                def _():
                    pltpu.sync_copy(table, out_h.at[cid, x * 2 + ch])

                plsc.subcore_barrier()

    return pl.kernel(
        body,
        out_type=jax.ShapeDtypeStruct((2, 6, N_T, 128), jnp.float32),
        mesh=_mesh(),
        compiler_params=pltpu.CompilerParams(needs_layout_passes=False),
        scratch_types=[
            pltpu.VMEM((128,), jnp.int32),
            pltpu.VMEM((128,), jnp.int32),
            pltpu.VMEM((128, 128), jnp.float32),
            pltpu.VMEM((128, 128), jnp.float32),
            pltpu.VMEM((128,), jnp.float32),
            pltpu.VMEM((128,), jnp.float32),
            pltpu.SemaphoreType.DMA,
            pltpu.SemaphoreType.DMA,
            pltpu.SemaphoreType.DMA,
            pltpu.SemaphoreType.DMA,
            pltpu.VMEM_SHARED((N_T, 128), jnp.float32),
        ],
    )(*coeff2, *rhat3, idx1d, zeros128)


def _edge1_body(n_valid, qi, qj, d, wa_t, wb_t, b_in, offs, coeff_s, w1i_t,
                w1j_t, w1f_t, w1d, b1, w2_t, b2, sem_t, semb, m_out, e_out):
    qi_v = qi[...]
    qj_v = qj[...]
    d_v = d[...]
    qe = qi_v @ wa_t[...] + qj_v @ wb_t[...] + b_in[...]
    rbf = jnp.exp(coeff_s[0, 0] * (d_v - offs[...]) ** 2)
    filt = rbf * qe
    pre = (qi_v @ w1i_t[...] + qj_v @ w1j_t[...] + filt @ w1f_t[...]
           + d_v * w1d[...] + b1[...])
    h = _silu(pre)
    m = h @ w2_t[...] + b2[...]
    att = _celu2(m @ sem_t[...] + semb[...])
    e = jnp.exp(jnp.minimum(att, 60.0))
    rows = (pl.program_id(0) * E_BLK
            + lax.broadcasted_iota(jnp.int32, (E_BLK, 1), 0))
    valid = rows < n_valid
    e128 = jnp.concatenate(
        [e, jnp.ones((E_BLK, 1), jnp.float32),
         jnp.zeros((E_BLK, 125), jnp.float32)], axis=1)
    m_out[...] = m
    e_out[...] = jnp.where(valid, e128, 0.0)


def _edge2_body(m, e128, ssg128, r, d, x0_t, x1_t, acat_out, c0_out, c1_out,
                r0_out, r1_out, r2_out):
    m_v = m[...]
    e = e128[...][:, 0:H]
    ssg = jnp.maximum(ssg128[...][:, 0:H], 1e-30)
    comb = e / ssg
    a0 = m_v * comb[:, 0:1]
    a1 = m_v * comb[:, 1:2]
    coeff = jnp.tanh(a0 @ x0_t[...] + a1 @ x1_t[...])
    acat_out[...] = jnp.concatenate([a0, a1], axis=1)
    c0_out[...] = coeff[:, 0:F]
    c1_out[...] = coeff[:, F:C]
    rhat = r[...] / (d[...] + 1e-05)
    r0_out[...] = rhat[:, 0:1]
    r1_out[...] = rhat[:, 1:2]
    r2_out[...] = rhat[:, 2:3]


def _node_body(*refs):
    (q, agg0a, agg0b, agg1a, agg1b), cs_refs = refs[0:5], refs[5:17]
    (cnt128, pw1_t, pb1, pw2_t, pb2, wq_t, wa_t, wc_t, nb1, nw2_t, nb2,
     out) = refs[17:]
    q_v = q[...]
    agg = jnp.concatenate(
        [agg0a[...][0, 0] + agg1a[...][0, 0],
         agg0b[...][0, 0] + agg1b[...][0, 0]], axis=1)
    cnt_v = jnp.maximum(cnt128[...][:, 2:3], 1.0)
    norm_parts = []
    for ch in range(2):
        acc = None
        for x in range(3):
            k = x * 2 + ch
            mean = (cs_refs[k][...][0, 0] + cs_refs[6 + k][...][0, 0]) / cnt_v
            sq = mean * mean
            acc = sq if acc is None else acc + sq
        norm_parts.append(acc)
    norm = jnp.concatenate(norm_parts, axis=1)
    qc = _silu(norm @ pw1_t[...] + pb1[...])
    qcomb = _silu(qc @ pw2_t[...] + pb2[...])
    h = _silu(q_v @ wq_t[...] + agg @ wa_t[...] + qcomb @ wc_t[...]
              + nb1[...])
    out[...] = q_v + _silu(h @ nw2_t[...] + nb2[...])


def _full_spec(shape):
    return pl.BlockSpec(shape, lambda i: (0,) * len(shape))


def _row_spec(blk, width):
    return pl.BlockSpec((blk, width), lambda i: (i, 0))


def kernel(q, mu, r_ij, d_ij, mlp_in_w, mlp_in_b, mlp_out_w1, mlp_out_b1,
           mlp_out_w2, mlp_out_b2, sem_w, sem_b, xmix_w, post_w1, post_b1,
           post_w2, post_b2, node_w1, node_b1, node_w2, node_b2, idx_i, idx_j):
    n_atoms = q.shape[0]
    n_pairs = idx_i.shape[0]
    f32 = jnp.float32

    pad_rbf = lambda a, axis: jnp.concatenate(
        [a, jnp.zeros(a.shape[:axis] + (RBF_PAD - N_RBF,) + a.shape[axis + 1:],
                      f32)], axis=axis)
    wa_t = pad_rbf(mlp_in_w[:, :F].T, 1)
    wb_t = pad_rbf(mlp_in_w[:, F:].T, 1)
    b_in = pad_rbf(mlp_in_b[None, :], 1)
    offsets = jnp.linspace(0.0, CUTOFF, N_RBF)
    offs = pad_rbf(offsets[None, :], 1)
    width = offsets[1] - offsets[0]
    coeff_s = jnp.full((1, 1), -0.5 / (width ** 2), f32)
    w1i_t = mlp_out_w1[:, :F].T
    w1j_t = mlp_out_w1[:, F:2 * F].T
    w1f_t = pad_rbf(mlp_out_w1[:, 2 * F:2 * F + N_RBF], 1).T
    w1d = mlp_out_w1[:, 2 * F + N_RBF][None, :]
    b1 = mlp_out_b1[None, :]
    w2_t = mlp_out_w2.T
    b2 = mlp_out_b2[None, :]
    sem_t = sem_w.T
    semb = sem_b[None, :]
    xm = xmix_w.reshape(C, F, H)
    x0_t = xm[:, :, 0].T
    x1_t = xm[:, :, 1].T
    pw1_t = post_w1.T
    pb1 = post_b1[None, :]
    pw2_t = post_w2.T
    pb2 = post_b2[None, :]
    nq_t = node_w1[:, :F].T
    na_perm = node_w1[:, F:F + C].reshape(F, F, H).transpose(2, 1, 0)
    na_t = na_perm.reshape(C, F)
    nc_t = node_w1[:, F + C:].T
    nb1 = node_b1[None, :]
    nw2_t = node_w2.T
    nb2 = node_b2[None, :]

    pad_e = E_P - n_pairs
    idx_i_p = jnp.concatenate([idx_i, jnp.zeros((pad_e,), jnp.int32)])
    idx_j_p = jnp.concatenate([idx_j, jnp.zeros((pad_e,), jnp.int32)])
    idxcat2d = jnp.concatenate([idx_i_p, idx_j_p]).reshape(2 * E_P // 128, 128)
    idxj2d = idx_j_p.reshape(E_P // 128, 128)
    d2 = d_ij.astype(f32)
    d_p = jnp.concatenate([d2, jnp.ones((pad_e, 1), f32)], axis=0)
    r_p = jnp.concatenate([r_ij, jnp.zeros((pad_e, 3), f32)], axis=0)
    zeros128 = jnp.zeros((N_T, 128), f32)

    grid_e = E_P // E_BLK

    gathered = _sc_gather(q, idxcat2d, 2 * E_P, F)

    qi_spec = pl.BlockSpec((E_BLK, F), lambda i: (i, 0))
    qj_spec = pl.BlockSpec((E_BLK, F), lambda i: (i + grid_e, 0))
    m, e128 = pl.pallas_call(
        functools.partial(_edge1_body, n_pairs),
        grid=(grid_e,),
        in_specs=[
            qi_spec, qj_spec, _row_spec(E_BLK, 1),
            _full_spec((F, RBF_PAD)), _full_spec((F, RBF_PAD)),
            _full_spec((1, RBF_PAD)), _full_spec((1, RBF_PAD)),
            _full_spec((1, 1)),
            _full_spec((F, F)), _full_spec((F, F)), _full_spec((RBF_PAD, F)),
            _full_spec((1, F)), _full_spec((1, F)), _full_spec((F, F)),
            _full_spec((1, F)), _full_spec((F, H)), _full_spec((1, H)),
        ],
        out_specs=[_row_spec(E_BLK, F), _row_spec(E_BLK, 128)],
        out_shape=[
            jax.ShapeDtypeStruct((E_P, F), f32),
            jax.ShapeDtypeStruct((E_P, 128), f32),
        ],

    )(gathered, gathered, d_p, wa_t, wb_t, b_in, offs, coeff_s, w1i_t, w1j_t,
      w1f_t, w1d, b1, w2_t, b2, sem_t, semb)

    parts = _sc_scatter_seg(e128, idx_j_p, zeros128)
    seg128 = parts[0] + parts[1]
    ssg128 = _sc_gather(seg128, idxj2d, E_P, 128)

    e2_outs = pl.pallas_call(
        _edge2_body,
        grid=(grid_e,),
        in_specs=[
            _row_spec(E_BLK, F), _row_spec(E_BLK, 128), _row_spec(E_BLK, 128),
            _row_spec(E_BLK, 3), _row_spec(E_BLK, 1),
            _full_spec((F, C)), _full_spec((F, C)),
        ],
        out_specs=([_row_spec(E_BLK, C)] + [_row_spec(E_BLK, F)] * 2
                   + [_row_spec(E_BLK, 1)] * 3),
        out_shape=([jax.ShapeDtypeStruct((E_P, C), f32)]
                   + [jax.ShapeDtypeStruct((E_P, F), f32)] * 2
                   + [jax.ShapeDtypeStruct((E_P, 1), f32)] * 3),

    )(m, e128, ssg128, r_p, d_p, x0_t, x1_t)
    acat = e2_outs[0]
    coeff2 = tuple(e2_outs[1:3])
    rhat3 = tuple(a.reshape(E_P) for a in e2_outs[3:6])

    aggp = _sc_scatter_acat(acat, idx_j_p, zeros128)
    csp = _sc_scatter_combos(coeff2, rhat3, idx_j_p,
                             zeros128)

    agg_specs = [
        pl.BlockSpec((1, 1, N_BLK, 128),
                     lambda i, c=c, ch=ch: (c, ch, i, 0))
        for c in range(2) for ch in range(2)
    ]
    cs_specs = [
        pl.BlockSpec((1, 1, N_BLK, 128), lambda i, c=c, k=k: (c, k, i, 0))
        for c in range(2) for k in range(6)
    ]

    grid_n = n_atoms // N_BLK
    out = pl.pallas_call(
        _node_body,
        grid=(grid_n,),
        in_specs=[_row_spec(N_BLK, F)] + agg_specs + cs_specs + [
            _row_spec(N_BLK, 128),
            _full_spec((C, F)), _full_spec((1, F)), _full_spec((F, F)),
            _full_spec((1, F)), _full_spec((F, F)), _full_spec((C, F)),
            _full_spec((F, F)), _full_spec((1, F)), _full_spec((F, F)),
            _full_spec((1, F)),
        ],
        out_specs=[_row_spec(N_BLK, F)],
        out_shape=[jax.ShapeDtypeStruct((n_atoms, F), f32)],

    )(q, aggp, aggp, aggp, aggp, *([csp] * 12), seg128, pw1_t, pb1, pw2_t,
      pb2, nq_t, na_t, nc_t, nb1, nw2_t, nb2)[0]
    return out
